# baseline (speedup 1.0000x reference)
import functools

import jax
import jax.numpy as jnp
from jax import lax
from jax.experimental import pallas as pl
from jax.experimental.pallas import tpu as pltpu
from jax.experimental.pallas import tpu_sc as plsc

N = 10000
N_PAD = 10240
E = 320000
E_PAD = 327680
NC = 2
NS = 16
NW = NC * NS
EPT = E_PAD // NW
CHUNK = 128
NCHUNK = EPT // CHUNK
NBUF = 4
ROWS_PT = N_PAD // NS
RB = 1024
GRID = N_PAD // RB

_MESH = plsc.VectorSubcoreMesh(core_axis_name="c", subcore_axis_name="s")
_PREC = lax.Precision.HIGHEST
_SC_PARAMS = pltpu.CompilerParams(use_tc_tiling_on_sc=False)


def _deg_kernel(col3, w3):
    @functools.partial(
        pl.kernel,
        mesh=_MESH,
        compiler_params=_SC_PARAMS,
        out_type=jax.ShapeDtypeStruct((NC, N_PAD), jnp.float32),
        scratch_types=[
            pltpu.VMEM((NCHUNK, CHUNK), jnp.int32),
            pltpu.VMEM((NCHUNK, CHUNK), jnp.float32),
            pltpu.VMEM((CHUNK,), jnp.float32),
            pltpu.VMEM_SHARED((N_PAD,), jnp.float32),
            pltpu.SemaphoreType.DMA,
            pltpu.SemaphoreType.DMA,
        ],
    )
    def deg(col_hbm, w_hbm, out_hbm, cidx2, w2, zbuf_v, acc_sh, s0, s1):
        cid = lax.axis_index("c")
        sid = lax.axis_index("s")
        wid = sid * NC + cid
        ld0 = pltpu.async_copy(col_hbm.at[wid], cidx2, s0)
        ld1 = pltpu.async_copy(w_hbm.at[wid], w2, s1)
        for j in range(CHUNK // 16):
            zbuf_v[pl.ds(16 * j, 16)] = jnp.zeros((16,), jnp.float32)
        r0 = sid * ROWS_PT
        for k in range(ROWS_PT // CHUNK):
            pltpu.sync_copy(zbuf_v, acc_sh.at[pl.ds(r0 + k * CHUNK, CHUNK)])
        ld0.wait()
        ld1.wait()
        plsc.subcore_barrier()

        ssem = (s0, s1)

        def scat(i, b):
            return pltpu.async_copy(w2.at[i], acc_sh.at[cidx2.at[i]],
                                    ssem[b], add=True)

        def swait(i, b):
            pltpu.make_async_copy(w2.at[i], acc_sh.at[cidx2.at[i]],
                                  ssem[b]).wait()

        scat(0, 0)
        scat(1, 1)

        def body(g, carry):
            i0 = 2 * g
            swait(i0 - 2, 0)
            scat(i0, 0)
            swait(i0 - 1, 1)
            scat(i0 + 1, 1)
            return carry

        lax.fori_loop(1, NCHUNK // 2, body, 0)
        swait(NCHUNK - 2, 0)
        swait(NCHUNK - 1, 1)
        plsc.subcore_barrier()
        for k in range(ROWS_PT // CHUNK):
            off = r0 + k * CHUNK
            pltpu.sync_copy(acc_sh.at[pl.ds(off, CHUNK)], zbuf_v)
            pltpu.sync_copy(zbuf_v, out_hbm.at[cid, pl.ds(off, CHUNK)])

    return deg(col3, w3)


def _prop(u, row3, col3, w3, C):
    @functools.partial(
        pl.kernel,
        mesh=_MESH,
        compiler_params=_SC_PARAMS,
        out_type=jax.ShapeDtypeStruct((NC, N_PAD, C), jnp.float32),
        scratch_types=[
            pltpu.VMEM((NCHUNK, CHUNK), jnp.int32),
            pltpu.VMEM((NCHUNK, CHUNK), jnp.int32),
            pltpu.VMEM((NCHUNK, CHUNK), jnp.float32),
            pltpu.VMEM((CHUNK, C), jnp.float32),
            pltpu.VMEM((CHUNK, C), jnp.float32),
            pltpu.VMEM((CHUNK, C), jnp.float32),
            pltpu.VMEM((CHUNK, C), jnp.float32),
            pltpu.VMEM((CHUNK, C), jnp.float32),
            pltpu.VMEM_SHARED((N_PAD, C), jnp.float32),
            pltpu.SemaphoreType.DMA,
            pltpu.SemaphoreType.DMA,
            pltpu.SemaphoreType.DMA,
            pltpu.SemaphoreType.DMA,
            pltpu.SemaphoreType.DMA,
            pltpu.SemaphoreType.DMA,
            pltpu.SemaphoreType.DMA,
            pltpu.SemaphoreType.DMA,
        ],
    )
    def prop(u_hbm, row_hbm, col_hbm, w_hbm, out_hbm,
             ridx2, cidx2, w2, b0, b1, b2, b3, zbuf, acc_sh,
             g0, g1, g2, g3, s0, s1, s2, s3):
        bufs = (b0, b1, b2, b3)
        gsem = (g0, g1, g2, g3)
        ssem = (s0, s1, s2, s3)
        cid = lax.axis_index("c")
        sid = lax.axis_index("s")
        wid = sid * NC + cid

        ld = [pltpu.async_copy(row_hbm.at[wid], ridx2, g0),
              pltpu.async_copy(col_hbm.at[wid], cidx2, g1),
              pltpu.async_copy(w_hbm.at[wid], w2, g2)]

        def zbody(e, carry):
            for j in range(C // 16):
                zbuf[e, pl.ds(16 * j, 16)] = jnp.zeros((16,), jnp.float32)
            return carry

        lax.fori_loop(0, CHUNK, zbody, 0)
        r0 = sid * ROWS_PT
        for k in range(ROWS_PT // CHUNK):
            pltpu.sync_copy(zbuf, acc_sh.at[pl.ds(r0 + k * CHUNK, CHUNK)])
        for h in ld:
            h.wait()
        plsc.subcore_barrier()

        def gather(i, b):
            pltpu.async_copy(u_hbm.at[ridx2.at[i]], bufs[b], gsem[b])

        def gwait(i, b):
            pltpu.make_async_copy(u_hbm.at[ridx2.at[i]], bufs[b],
                                  gsem[b]).wait()

        def scat(i, b):
            pltpu.async_copy(bufs[b], acc_sh.at[cidx2.at[i]], ssem[b],
                             add=True)

        def swait(i, b):
            pltpu.make_async_copy(bufs[b], acc_sh.at[cidx2.at[i]],
                                  ssem[b]).wait()

        def mul(i, b):
            buf = bufs[b]

            def mbody(g, carry):
                wv = w2[i, pl.ds(16 * g, 16)]
                for k in range(16):
                    ws = wv[k]
                    for j in range(C // 16):
                        sl = pl.ds(16 * j, 16)
                        buf[16 * g + k, sl] = buf[16 * g + k, sl] * ws
                return carry

            lax.fori_loop(0, CHUNK // 16, mbody, 0)

        for j in range(3):
            gather(j, j)
        for i in range(NBUF):
            gwait(i, i)
            mul(i, i)
            scat(i, i)
            if i >= 1:
                swait(i - 1, i - 1)
            gather(i + 3, (i + 3) % NBUF)

        def body(g, carry):
            for b in range(NBUF):
                i = NBUF * g + b
                gwait(i, b)
                mul(i, b)
                scat(i, b)
                pb = (b + 3) % NBUF
                swait(i - 1, pb)
                i3 = i + 3

                @pl.when(i3 < NCHUNK)
                def _():
                    gather(i3, pb)

            return carry

        lax.fori_loop(1, NCHUNK // NBUF, body, 0)
        swait(NCHUNK - 1, (NCHUNK - 1) % NBUF)
        plsc.subcore_barrier()

        def acc_sl(k):
            return acc_sh.at[pl.ds(r0 + k * CHUNK, CHUNK)]

        def out_sl(k):
            return out_hbm.at[cid, pl.ds(r0 + k * CHUNK, CHUNK)]

        outs = []
        for k in range(5):
            b = k % 4
            if k >= 4:
                outs[b].wait()
            pltpu.sync_copy(acc_sl(k), bufs[b])
            outs.append(pltpu.async_copy(bufs[b], out_sl(k), ssem[b]))
        for k in range(1, 5):
            outs[k].wait()

    return prop(u, row3, col3, w3)


def _prop_ring(u, row2, col2, w2, C, nbuf, cmul=None):
    U = 8
    D = nbuf - 1
    if cmul is None:
        cmul = C

    @functools.partial(
        pl.kernel,
        mesh=_MESH,
        compiler_params=_SC_PARAMS,
        out_type=jax.ShapeDtypeStruct((NC, N_PAD, C), jnp.float32),
        scratch_types=[
            [pltpu.VMEM((CHUNK,), jnp.int32) for _ in range(U)],
            [pltpu.VMEM((CHUNK,), jnp.int32) for _ in range(U)],
            [pltpu.VMEM((CHUNK,), jnp.float32) for _ in range(U)],
            [pltpu.VMEM((CHUNK, C), jnp.float32) for _ in range(nbuf)],
            pltpu.VMEM_SHARED((N_PAD, C), jnp.float32),
            [pltpu.SemaphoreType.DMA for _ in range(U)],
            [pltpu.SemaphoreType.DMA for _ in range(nbuf)],
            [pltpu.SemaphoreType.DMA for _ in range(nbuf)],
        ],
    )
    def prop(u_hbm, row_hbm, col_hbm, w_hbm, out_hbm,
             ridx, cidx, wv8, bufs, acc_sh, isem, gsem, ssem):
        cid = lax.axis_index("c")
        sid = lax.axis_index("s")
        wid = sid * NC + cid
        base = wid * NCHUNK

        def iload(i, s):
            pltpu.async_copy(row_hbm.at[base + i], ridx[s], isem[s])
            pltpu.async_copy(col_hbm.at[base + i], cidx[s], isem[s])
            pltpu.async_copy(w_hbm.at[base + i], wv8[s], isem[s])

        def iwait(i, s):
            pltpu.make_async_copy(row_hbm.at[base + i], ridx[s],
                                  isem[s]).wait()
            pltpu.make_async_copy(col_hbm.at[base + i], cidx[s],
                                  isem[s]).wait()
            pltpu.make_async_copy(w_hbm.at[base + i], wv8[s], isem[s]).wait()

        def gather(i, b, s):
            pltpu.async_copy(u_hbm.at[ridx[s]], bufs[b], gsem[b])

        def gwait(i, b, s):
            pltpu.make_async_copy(u_hbm.at[ridx[s]], bufs[b], gsem[b]).wait()

        def scat(i, b, s):
            pltpu.async_copy(bufs[b], acc_sh.at[cidx[s]], ssem[b], add=True)

        def swait(i, b, s):
            pltpu.make_async_copy(bufs[b], acc_sh.at[cidx[s]],
                                  ssem[b]).wait()

        def mul(i, b, s):
            buf = bufs[b]

            def mbody(g, carry):
                wvec = wv8[s][pl.ds(16 * g, 16)]
                for k in range(16):
                    ws = wvec[k]
                    for j in range(cmul // 16):
                        sl = pl.ds(16 * j, 16)
                        buf[16 * g + k, sl] = buf[16 * g + k, sl] * ws
                return carry

            lax.fori_loop(0, CHUNK // 16, mbody, 0)

        def zbody(e, carry):
            for j in range(C // 16):
                bufs[0][e, pl.ds(16 * j, 16)] = jnp.zeros((16,), jnp.float32)
            return carry

        lax.fori_loop(0, CHUNK, zbody, 0)
        r0 = sid * ROWS_PT
        for k in range(ROWS_PT // CHUNK):
            pltpu.sync_copy(bufs[0], acc_sh.at[pl.ds(r0 + k * CHUNK, CHUNK)])
        plsc.subcore_barrier()

        def step(i, bs, static):
            bb = bs % nbuf
            gwait(i, bb, bs)
            if (not static) or bs >= 1:
                swait(i - 1, (bs - 1) % nbuf, (bs - 1) % U)
            nd = i + D
            nd_b, nd_s = (bs + D) % nbuf, (bs + D) % U
            ni = i + D + 2
            ni_s = (bs + D + 2) % U

            def do_gather():
                iwait(nd, nd_s)
                gather(nd, nd_b, nd_s)

            def do_iload():
                iload(ni, ni_s)

            if static:
                if nd < NCHUNK:
                    do_gather()
                if ni < NCHUNK:
                    do_iload()
            else:
                @pl.when(nd < NCHUNK)
                def _():
                    do_gather()

                @pl.when(ni < NCHUNK)
                def _():
                    do_iload()

            mul(i, bb, bs)
            scat(i, bb, bs)

        for j in range(D + 2):
            iload(j, j % U)
        for j in range(D):
            iwait(j, j % U)
            gather(j, j % nbuf, j % U)
        for i in range(U):
            step(i, i, static=True)

        def body(g, carry):
            for b in range(U):
                step(U * g + b, b, static=False)
            return carry

        lax.fori_loop(1, NCHUNK // U, body, 0)
        swait(NCHUNK - 1, (NCHUNK - 1) % nbuf, (NCHUNK - 1) % U)
        plsc.subcore_barrier()

        def acc_sl(k):
            return acc_sh.at[pl.ds(r0 + k * CHUNK, CHUNK)]

        def out_sl(k):
            return out_hbm.at[cid, pl.ds(r0 + k * CHUNK, CHUNK)]

        outs = []
        for k in range(5):
            b = k % nbuf
            if k >= nbuf:
                outs[k - nbuf].wait()
            pltpu.sync_copy(acc_sl(k), bufs[b])
            outs.append(pltpu.async_copy(bufs[b], out_sl(k), ssem[b]))
        for k in range(max(0, 5 - nbuf), 5):
            outs[k].wait()

    return prop(u, row2, col2, w2)


def _dinv_block(degp):
    deg = degp[0] + degp[1]
    safe = jnp.where(deg > 0, deg, 1.0)
    return jnp.where(deg > 0, lax.rsqrt(safe), 0.0).reshape(-1, 1)


def _tc_head(h, W, b, degp):
    fi, fo = W.shape

    def body(h_ref, w_ref, b_ref, degp_ref, out_ref, u_ref):
        dinv = _dinv_block(degp_ref[...])
        hv = h_ref[...]
        out_ref[...] = (
            jnp.dot(hv, w_ref[...], preferred_element_type=jnp.float32,
                    precision=_PREC) + b_ref[...])
        u_ref[...] = hv * dinv

    return pl.pallas_call(
        body,
        grid=(GRID,),
        in_specs=[
            pl.BlockSpec((RB, fi), lambda i: (i, 0)),
            pl.BlockSpec((fi, fo), lambda i: (0, 0)),
            pl.BlockSpec((1, fo), lambda i: (0, 0)),
            pl.BlockSpec((2, RB), lambda i: (0, i)),
        ],
        out_specs=[
            pl.BlockSpec((RB, fo), lambda i: (i, 0)),
            pl.BlockSpec((RB, fi), lambda i: (i, 0)),
        ],
        out_shape=[
            jax.ShapeDtypeStruct((N_PAD, fo), jnp.float32),
            jax.ShapeDtypeStruct((N_PAD, fi), jnp.float32),
        ],
    )(h, W, b.reshape(1, -1), degp)


def _tc_mid(P, W, b, degp, out_in):
    fi, fo = W.shape

    def body(p_ref, w_ref, b_ref, degp_ref, oin_ref, out_ref, u_ref):
        dinv = _dinv_block(degp_ref[...])
        hk = (p_ref[0] + p_ref[1]) * dinv
        out_ref[...] = oin_ref[...] + (
            jnp.dot(hk, w_ref[...], preferred_element_type=jnp.float32,
                    precision=_PREC) + b_ref[...])
        u_ref[...] = hk * dinv

    return pl.pallas_call(
        body,
        grid=(GRID,),
        in_specs=[
            pl.BlockSpec((2, RB, fi), lambda i: (0, i, 0)),
            pl.BlockSpec((fi, fo), lambda i: (0, 0)),
            pl.BlockSpec((1, fo), lambda i: (0, 0)),
            pl.BlockSpec((2, RB), lambda i: (0, i)),
            pl.BlockSpec((RB, fo), lambda i: (i, 0)),
        ],
        out_specs=[
            pl.BlockSpec((RB, fo), lambda i: (i, 0)),
            pl.BlockSpec((RB, fi), lambda i: (i, 0)),
        ],
        out_shape=[
            jax.ShapeDtypeStruct((N_PAD, fo), jnp.float32),
            jax.ShapeDtypeStruct((N_PAD, fi), jnp.float32),
        ],
    )(P, W, b.reshape(1, -1), degp, out_in)


def _tc_bridge(P, W, b, degp, out_in, Wn, bn):
    fi, fo = W.shape
    fon = Wn.shape[1]

    def body(p_ref, w_ref, b_ref, degp_ref, oin_ref, wn_ref, bn_ref,
             out_ref, u_ref):
        dinv = _dinv_block(degp_ref[...])
        hk = (p_ref[0] + p_ref[1]) * dinv
        hp = jax.nn.relu(
            oin_ref[...] + jnp.dot(hk, w_ref[...],
                                   preferred_element_type=jnp.float32,
                                   precision=_PREC) + b_ref[...])
        out_ref[...] = (
            jnp.dot(hp, wn_ref[...], preferred_element_type=jnp.float32,
                    precision=_PREC) + bn_ref[...])
        u_ref[...] = hp * dinv

    return pl.pallas_call(
        body,
        grid=(GRID,),
        in_specs=[
            pl.BlockSpec((2, RB, fi), lambda i: (0, i, 0)),
            pl.BlockSpec((fi, fo), lambda i: (0, 0)),
            pl.BlockSpec((1, fo), lambda i: (0, 0)),
            pl.BlockSpec((2, RB), lambda i: (0, i)),
            pl.BlockSpec((RB, fo), lambda i: (i, 0)),
            pl.BlockSpec((fo, fon), lambda i: (0, 0)),
            pl.BlockSpec((1, fon), lambda i: (0, 0)),
        ],
        out_specs=[
            pl.BlockSpec((RB, fon), lambda i: (i, 0)),
            pl.BlockSpec((RB, fo), lambda i: (i, 0)),
        ],
        out_shape=[
            jax.ShapeDtypeStruct((N_PAD, fon), jnp.float32),
            jax.ShapeDtypeStruct((N_PAD, fo), jnp.float32),
        ],
    )(P, W, b.reshape(1, -1), degp, out_in, Wn, bn.reshape(1, -1))


def _tc_tail(P, W, b, degp, out_in, fcW, fcb):
    fi, fo = W.shape
    fon = fcW.shape[1]

    def body(p_ref, w_ref, b_ref, degp_ref, oin_ref, wn_ref, bn_ref, out_ref):
        dinv = _dinv_block(degp_ref[...])
        hk = (p_ref[0] + p_ref[1]) * dinv
        hp = jax.nn.relu(
            oin_ref[...] + jnp.dot(hk, w_ref[...],
                                   preferred_element_type=jnp.float32,
                                   precision=_PREC) + b_ref[...])
        z = (jnp.dot(hp, wn_ref[...], preferred_element_type=jnp.float32,
                     precision=_PREC) + bn_ref[...])
        nrm = jnp.sqrt(jnp.sum(z * z, axis=-1, keepdims=True))
        out_ref[...] = z / jnp.maximum(nrm, 1e-12)

    return pl.pallas_call(
        body,
        grid=(GRID,),
        in_specs=[
            pl.BlockSpec((2, RB, fi), lambda i: (0, i, 0)),
            pl.BlockSpec((fi, fo), lambda i: (0, 0)),
            pl.BlockSpec((1, fo), lambda i: (0, 0)),
            pl.BlockSpec((2, RB), lambda i: (0, i)),
            pl.BlockSpec((RB, fo), lambda i: (i, 0)),
            pl.BlockSpec((fo, fon), lambda i: (0, 0)),
            pl.BlockSpec((1, fon), lambda i: (0, 0)),
        ],
        out_specs=pl.BlockSpec((RB, fon), lambda i: (i, 0)),
        out_shape=jax.ShapeDtypeStruct((N_PAD, fon), jnp.float32),
    )(P, W, b.reshape(1, -1), degp, out_in, fcW, fcb.reshape(1, -1))


def kernel(x, edge_index, edge_attr,
           W1_0, b1_0, W1_1, b1_1,
           W2_0, b2_0, W2_1, b2_1, W2_2, b2_2,
           W3_0, b3_0, W3_1, b3_1, W3_2, b3_2, W3_3, b3_3,
           fc_W, fc_b):
    npad = E_PAD - E
    pad_nodes = (jnp.arange(npad, dtype=jnp.int32) % (N_PAD - N)) + N
    row3 = jnp.concatenate([edge_index[0], pad_nodes]).reshape(
        NW, NCHUNK, CHUNK)
    col3 = jnp.concatenate([edge_index[1], pad_nodes]).reshape(
        NW, NCHUNK, CHUNK)
    w3 = jnp.concatenate(
        [edge_attr, jnp.zeros((npad,), jnp.float32)]).reshape(
        NW, NCHUNK, CHUNK)
    x_p = jnp.pad(x, ((0, N_PAD - N), (0, 0)))

    row2 = row3.reshape(-1, CHUNK)
    col2 = col3.reshape(-1, CHUNK)
    w2 = w3.reshape(-1, CHUNK)

    degp = _deg_kernel(col3, w3)

    out0, u = _tc_head(x_p, W1_0, b1_0, degp)
    P = _prop_ring(u, row2, col2, w2, 128, 2)
    out0, u = _tc_bridge(P, W1_1, b1_1, degp, out0, W2_0, b2_0)
    P = _prop(u, row3, col3, w3, 32)
    out0, u = _tc_mid(P, W2_1, b2_1, degp, out0)
    P = _prop(u, row3, col3, w3, 32)
    def prop64(u):
        u72 = jnp.pad(u, ((0, 0), (0, 8)))
        return _prop_ring(u72, row2, col2, w2, 72, 4, cmul=64)[:, :, :64]

    out0, u = _tc_bridge(P, W2_2, b2_2, degp, out0, W3_0, b3_0)
    P = prop64(u)
    out0, u = _tc_mid(P, W3_1, b3_1, degp, out0)
    P = prop64(u)
    out0, u = _tc_mid(P, W3_2, b3_2, degp, out0)
    P = prop64(u)
    final = _tc_tail(P, W3_3, b3_3, degp, out0, fc_W, fc_b)
    return final[:N]


# speedup vs baseline: 1.1316x; 1.1316x over previous
import functools

import jax
import jax.numpy as jnp
from jax import lax
from jax.experimental import pallas as pl
from jax.experimental.pallas import tpu as pltpu
from jax.experimental.pallas import tpu_sc as plsc

N = 10000
N_PAD = 10240
E = 320000
E_PAD = 327680
NC = 2
NS = 16
NW = NC * NS
EPT = E_PAD // NW
CHUNK = 128
NCHUNK = EPT // CHUNK
NBUF = 4
ROWS_PT = N_PAD // NS
RB = 1024
GRID = N_PAD // RB

_MESH = plsc.VectorSubcoreMesh(core_axis_name="c", subcore_axis_name="s")
_PREC = lax.Precision.HIGHEST
_SC_PARAMS = pltpu.CompilerParams(use_tc_tiling_on_sc=False)


def _deg_kernel(col3, w3):
    @functools.partial(
        pl.kernel,
        mesh=_MESH,
        compiler_params=_SC_PARAMS,
        out_type=jax.ShapeDtypeStruct((NC, N_PAD), jnp.float32),
        scratch_types=[
            pltpu.VMEM((NCHUNK, CHUNK), jnp.int32),
            pltpu.VMEM((NCHUNK, CHUNK), jnp.float32),
            pltpu.VMEM((CHUNK,), jnp.float32),
            pltpu.VMEM_SHARED((N_PAD,), jnp.float32),
            pltpu.SemaphoreType.DMA,
            pltpu.SemaphoreType.DMA,
        ],
    )
    def deg(col_hbm, w_hbm, out_hbm, cidx2, w2, zbuf_v, acc_sh, s0, s1):
        cid = lax.axis_index("c")
        sid = lax.axis_index("s")
        wid = sid * NC + cid
        ld0 = pltpu.async_copy(col_hbm.at[wid], cidx2, s0)
        ld1 = pltpu.async_copy(w_hbm.at[wid], w2, s1)
        for j in range(CHUNK // 16):
            zbuf_v[pl.ds(16 * j, 16)] = jnp.zeros((16,), jnp.float32)
        r0 = sid * ROWS_PT
        for k in range(ROWS_PT // CHUNK):
            pltpu.sync_copy(zbuf_v, acc_sh.at[pl.ds(r0 + k * CHUNK, CHUNK)])
        ld0.wait()
        ld1.wait()
        plsc.subcore_barrier()

        ssem = (s0, s1)

        def scat(i, b):
            return pltpu.async_copy(w2.at[i], acc_sh.at[cidx2.at[i]],
                                    ssem[b], add=True)

        def swait(i, b):
            pltpu.make_async_copy(w2.at[i], acc_sh.at[cidx2.at[i]],
                                  ssem[b]).wait()

        scat(0, 0)
        scat(1, 1)

        def body(g, carry):
            i0 = 2 * g
            swait(i0 - 2, 0)
            scat(i0, 0)
            swait(i0 - 1, 1)
            scat(i0 + 1, 1)
            return carry

        lax.fori_loop(1, NCHUNK // 2, body, 0)
        swait(NCHUNK - 2, 0)
        swait(NCHUNK - 1, 1)
        plsc.subcore_barrier()
        for k in range(ROWS_PT // CHUNK):
            off = r0 + k * CHUNK
            pltpu.sync_copy(acc_sh.at[pl.ds(off, CHUNK)], zbuf_v)
            pltpu.sync_copy(zbuf_v, out_hbm.at[cid, pl.ds(off, CHUNK)])

    return deg(col3, w3)


def _prop(u, row3, col3, w3, C):
    @functools.partial(
        pl.kernel,
        mesh=_MESH,
        compiler_params=_SC_PARAMS,
        out_type=jax.ShapeDtypeStruct((NC, N_PAD, C), jnp.float32),
        scratch_types=[
            pltpu.VMEM((NCHUNK, CHUNK), jnp.int32),
            pltpu.VMEM((NCHUNK, CHUNK), jnp.int32),
            pltpu.VMEM((NCHUNK, CHUNK), jnp.float32),
            pltpu.VMEM((CHUNK, C), jnp.float32),
            pltpu.VMEM((CHUNK, C), jnp.float32),
            pltpu.VMEM((CHUNK, C), jnp.float32),
            pltpu.VMEM((CHUNK, C), jnp.float32),
            pltpu.VMEM((CHUNK, C), jnp.float32),
            pltpu.VMEM_SHARED((N_PAD, C), jnp.float32),
            pltpu.SemaphoreType.DMA,
            pltpu.SemaphoreType.DMA,
            pltpu.SemaphoreType.DMA,
            pltpu.SemaphoreType.DMA,
            pltpu.SemaphoreType.DMA,
            pltpu.SemaphoreType.DMA,
            pltpu.SemaphoreType.DMA,
            pltpu.SemaphoreType.DMA,
        ],
    )
    def prop(u_hbm, row_hbm, col_hbm, w_hbm, out_hbm,
             ridx2, cidx2, w2, b0, b1, b2, b3, zbuf, acc_sh,
             g0, g1, g2, g3, s0, s1, s2, s3):
        bufs = (b0, b1, b2, b3)
        gsem = (g0, g1, g2, g3)
        ssem = (s0, s1, s2, s3)
        cid = lax.axis_index("c")
        sid = lax.axis_index("s")
        wid = sid * NC + cid

        ld = [pltpu.async_copy(row_hbm.at[wid], ridx2, g0),
              pltpu.async_copy(col_hbm.at[wid], cidx2, g1),
              pltpu.async_copy(w_hbm.at[wid], w2, g2)]

        def zbody(e, carry):
            for j in range(C // 16):
                zbuf[e, pl.ds(16 * j, 16)] = jnp.zeros((16,), jnp.float32)
            return carry

        lax.fori_loop(0, CHUNK, zbody, 0)
        r0 = sid * ROWS_PT
        for k in range(ROWS_PT // CHUNK):
            pltpu.sync_copy(zbuf, acc_sh.at[pl.ds(r0 + k * CHUNK, CHUNK)])
        for h in ld:
            h.wait()
        plsc.subcore_barrier()

        def gather(i, b):
            pltpu.async_copy(u_hbm.at[ridx2.at[i]], bufs[b], gsem[b])

        def gwait(i, b):
            pltpu.make_async_copy(u_hbm.at[ridx2.at[i]], bufs[b],
                                  gsem[b]).wait()

        def scat(i, b):
            pltpu.async_copy(bufs[b], acc_sh.at[cidx2.at[i]], ssem[b],
                             add=True)

        def swait(i, b):
            pltpu.make_async_copy(bufs[b], acc_sh.at[cidx2.at[i]],
                                  ssem[b]).wait()

        def mul(i, b):
            buf = bufs[b]

            def mbody(g, carry):
                wv = w2[i, pl.ds(16 * g, 16)]
                for k in range(16):
                    ws = wv[k]
                    for j in range(C // 16):
                        sl = pl.ds(16 * j, 16)
                        buf[16 * g + k, sl] = buf[16 * g + k, sl] * ws
                return carry

            lax.fori_loop(0, CHUNK // 16, mbody, 0)

        for j in range(3):
            gather(j, j)
        for i in range(NBUF):
            gwait(i, i)
            mul(i, i)
            scat(i, i)
            if i >= 1:
                swait(i - 1, i - 1)
            gather(i + 3, (i + 3) % NBUF)

        def body(g, carry):
            for b in range(NBUF):
                i = NBUF * g + b
                gwait(i, b)
                mul(i, b)
                scat(i, b)
                pb = (b + 3) % NBUF
                swait(i - 1, pb)
                i3 = i + 3

                @pl.when(i3 < NCHUNK)
                def _():
                    gather(i3, pb)

            return carry

        lax.fori_loop(1, NCHUNK // NBUF, body, 0)
        swait(NCHUNK - 1, (NCHUNK - 1) % NBUF)
        plsc.subcore_barrier()

        def acc_sl(k):
            return acc_sh.at[pl.ds(r0 + k * CHUNK, CHUNK)]

        def out_sl(k):
            return out_hbm.at[cid, pl.ds(r0 + k * CHUNK, CHUNK)]

        outs = []
        for k in range(5):
            b = k % 4
            if k >= 4:
                outs[b].wait()
            pltpu.sync_copy(acc_sl(k), bufs[b])
            outs.append(pltpu.async_copy(bufs[b], out_sl(k), ssem[b]))
        for k in range(1, 5):
            outs[k].wait()

    return prop(u, row3, col3, w3)


def _prop_ring(u, row2, col2, w2, C, nbuf, cmul=None):
    U = 8
    D = nbuf - 1
    if cmul is None:
        cmul = C

    @functools.partial(
        pl.kernel,
        mesh=_MESH,
        compiler_params=_SC_PARAMS,
        out_type=jax.ShapeDtypeStruct((NC, N_PAD, C), jnp.float32),
        scratch_types=[
            [pltpu.VMEM((CHUNK,), jnp.int32) for _ in range(U)],
            [pltpu.VMEM((CHUNK,), jnp.int32) for _ in range(U)],
            [pltpu.VMEM((CHUNK,), jnp.float32) for _ in range(U)],
            [pltpu.VMEM((CHUNK, C), jnp.float32) for _ in range(nbuf)],
            pltpu.VMEM_SHARED((N_PAD, C), jnp.float32),
            [pltpu.SemaphoreType.DMA for _ in range(U)],
            [pltpu.SemaphoreType.DMA for _ in range(nbuf)],
            [pltpu.SemaphoreType.DMA for _ in range(nbuf)],
        ],
    )
    def prop(u_hbm, row_hbm, col_hbm, w_hbm, out_hbm,
             ridx, cidx, wv8, bufs, acc_sh, isem, gsem, ssem):
        cid = lax.axis_index("c")
        sid = lax.axis_index("s")
        wid = sid * NC + cid
        base = wid * NCHUNK

        def iload(i, s):
            pltpu.async_copy(row_hbm.at[base + i], ridx[s], isem[s])
            pltpu.async_copy(col_hbm.at[base + i], cidx[s], isem[s])
            pltpu.async_copy(w_hbm.at[base + i], wv8[s], isem[s])

        def iwait(i, s):
            pltpu.make_async_copy(row_hbm.at[base + i], ridx[s],
                                  isem[s]).wait()
            pltpu.make_async_copy(col_hbm.at[base + i], cidx[s],
                                  isem[s]).wait()
            pltpu.make_async_copy(w_hbm.at[base + i], wv8[s], isem[s]).wait()

        def gather(i, b, s):
            pltpu.async_copy(u_hbm.at[ridx[s]], bufs[b], gsem[b])

        def gwait(i, b, s):
            pltpu.make_async_copy(u_hbm.at[ridx[s]], bufs[b], gsem[b]).wait()

        def scat(i, b, s):
            pltpu.async_copy(bufs[b], acc_sh.at[cidx[s]], ssem[b], add=True)

        def swait(i, b, s):
            pltpu.make_async_copy(bufs[b], acc_sh.at[cidx[s]],
                                  ssem[b]).wait()

        def mul(i, b, s):
            buf = bufs[b]

            def mbody(g, carry):
                wvec = wv8[s][pl.ds(16 * g, 16)]
                for k in range(16):
                    ws = wvec[k]
                    for j in range(cmul // 16):
                        sl = pl.ds(16 * j, 16)
                        buf[16 * g + k, sl] = buf[16 * g + k, sl] * ws
                return carry

            lax.fori_loop(0, CHUNK // 16, mbody, 0)

        def zbody(e, carry):
            for j in range(C // 16):
                bufs[0][e, pl.ds(16 * j, 16)] = jnp.zeros((16,), jnp.float32)
            return carry

        lax.fori_loop(0, CHUNK, zbody, 0)
        r0 = sid * ROWS_PT
        for k in range(ROWS_PT // CHUNK):
            pltpu.sync_copy(bufs[0], acc_sh.at[pl.ds(r0 + k * CHUNK, CHUNK)])
        plsc.subcore_barrier()

        def step(i, bs, static):
            bb = bs % nbuf
            gwait(i, bb, bs)
            if (not static) or bs >= 1:
                swait(i - 1, (bs - 1) % nbuf, (bs - 1) % U)
            nd = i + D
            nd_b, nd_s = (bs + D) % nbuf, (bs + D) % U
            ni = i + D + 2
            ni_s = (bs + D + 2) % U

            def do_gather():
                iwait(nd, nd_s)
                gather(nd, nd_b, nd_s)

            def do_iload():
                iload(ni, ni_s)

            if static:
                if nd < NCHUNK:
                    do_gather()
                if ni < NCHUNK:
                    do_iload()
            else:
                @pl.when(nd < NCHUNK)
                def _():
                    do_gather()

                @pl.when(ni < NCHUNK)
                def _():
                    do_iload()

            mul(i, bb, bs)
            scat(i, bb, bs)

        for j in range(D + 2):
            iload(j, j % U)
        for j in range(D):
            iwait(j, j % U)
            gather(j, j % nbuf, j % U)
        for i in range(U):
            step(i, i, static=True)

        def body(g, carry):
            for b in range(U):
                step(U * g + b, b, static=False)
            return carry

        lax.fori_loop(1, NCHUNK // U, body, 0)
        swait(NCHUNK - 1, (NCHUNK - 1) % nbuf, (NCHUNK - 1) % U)
        plsc.subcore_barrier()

        def acc_sl(k):
            return acc_sh.at[pl.ds(r0 + k * CHUNK, CHUNK)]

        def out_sl(k):
            return out_hbm.at[cid, pl.ds(r0 + k * CHUNK, CHUNK)]

        outs = []
        for k in range(5):
            b = k % nbuf
            if k >= nbuf:
                outs[k - nbuf].wait()
            pltpu.sync_copy(acc_sl(k), bufs[b])
            outs.append(pltpu.async_copy(bufs[b], out_sl(k), ssem[b]))
        for k in range(max(0, 5 - nbuf), 5):
            outs[k].wait()

    return prop(u, row2, col2, w2)


def _dinv_block(degp):
    deg = degp[0] + degp[1]
    safe = jnp.where(deg > 0, deg, 1.0)
    return jnp.where(deg > 0, lax.rsqrt(safe), 0.0).reshape(-1, 1)


def _tc_head(h, W, b, degp):
    fi, fo = W.shape

    def body(h_ref, w_ref, b_ref, degp_ref, out_ref, u_ref):
        dinv = _dinv_block(degp_ref[...])
        hv = h_ref[...]
        out_ref[...] = (
            jnp.dot(hv, w_ref[...], preferred_element_type=jnp.float32,
                    precision=_PREC) + b_ref[...])
        u_ref[...] = hv * dinv

    return pl.pallas_call(
        body,
        grid=(GRID,),
        in_specs=[
            pl.BlockSpec((RB, fi), lambda i: (i, 0)),
            pl.BlockSpec((fi, fo), lambda i: (0, 0)),
            pl.BlockSpec((1, fo), lambda i: (0, 0)),
            pl.BlockSpec((2, RB), lambda i: (0, i)),
        ],
        out_specs=[
            pl.BlockSpec((RB, fo), lambda i: (i, 0)),
            pl.BlockSpec((RB, fi), lambda i: (i, 0)),
        ],
        out_shape=[
            jax.ShapeDtypeStruct((N_PAD, fo), jnp.float32),
            jax.ShapeDtypeStruct((N_PAD, fi), jnp.float32),
        ],
    )(h, W, b.reshape(1, -1), degp)


def _tc_mid(P, W, b, degp, out_in):
    fi, fo = W.shape

    def body(p_ref, w_ref, b_ref, degp_ref, oin_ref, out_ref, u_ref):
        dinv = _dinv_block(degp_ref[...])
        hk = (p_ref[0] + p_ref[1]) * dinv
        out_ref[...] = oin_ref[...] + (
            jnp.dot(hk, w_ref[...], preferred_element_type=jnp.float32,
                    precision=_PREC) + b_ref[...])
        u_ref[...] = hk * dinv

    return pl.pallas_call(
        body,
        grid=(GRID,),
        in_specs=[
            pl.BlockSpec((2, RB, fi), lambda i: (0, i, 0)),
            pl.BlockSpec((fi, fo), lambda i: (0, 0)),
            pl.BlockSpec((1, fo), lambda i: (0, 0)),
            pl.BlockSpec((2, RB), lambda i: (0, i)),
            pl.BlockSpec((RB, fo), lambda i: (i, 0)),
        ],
        out_specs=[
            pl.BlockSpec((RB, fo), lambda i: (i, 0)),
            pl.BlockSpec((RB, fi), lambda i: (i, 0)),
        ],
        out_shape=[
            jax.ShapeDtypeStruct((N_PAD, fo), jnp.float32),
            jax.ShapeDtypeStruct((N_PAD, fi), jnp.float32),
        ],
    )(P, W, b.reshape(1, -1), degp, out_in)


def _tc_bridge(P, W, b, degp, out_in, Wn, bn):
    fi, fo = W.shape
    fon = Wn.shape[1]

    def body(p_ref, w_ref, b_ref, degp_ref, oin_ref, wn_ref, bn_ref,
             out_ref, u_ref):
        dinv = _dinv_block(degp_ref[...])
        hk = (p_ref[0] + p_ref[1]) * dinv
        hp = jax.nn.relu(
            oin_ref[...] + jnp.dot(hk, w_ref[...],
                                   preferred_element_type=jnp.float32,
                                   precision=_PREC) + b_ref[...])
        out_ref[...] = (
            jnp.dot(hp, wn_ref[...], preferred_element_type=jnp.float32,
                    precision=_PREC) + bn_ref[...])
        u_ref[...] = hp * dinv

    return pl.pallas_call(
        body,
        grid=(GRID,),
        in_specs=[
            pl.BlockSpec((2, RB, fi), lambda i: (0, i, 0)),
            pl.BlockSpec((fi, fo), lambda i: (0, 0)),
            pl.BlockSpec((1, fo), lambda i: (0, 0)),
            pl.BlockSpec((2, RB), lambda i: (0, i)),
            pl.BlockSpec((RB, fo), lambda i: (i, 0)),
            pl.BlockSpec((fo, fon), lambda i: (0, 0)),
            pl.BlockSpec((1, fon), lambda i: (0, 0)),
        ],
        out_specs=[
            pl.BlockSpec((RB, fon), lambda i: (i, 0)),
            pl.BlockSpec((RB, fo), lambda i: (i, 0)),
        ],
        out_shape=[
            jax.ShapeDtypeStruct((N_PAD, fon), jnp.float32),
            jax.ShapeDtypeStruct((N_PAD, fo), jnp.float32),
        ],
    )(P, W, b.reshape(1, -1), degp, out_in, Wn, bn.reshape(1, -1))


def _tc_tail(P, W, b, degp, out_in, fcW, fcb):
    fi, fo = W.shape
    fon = fcW.shape[1]

    def body(p_ref, w_ref, b_ref, degp_ref, oin_ref, wn_ref, bn_ref, out_ref):
        dinv = _dinv_block(degp_ref[...])
        hk = (p_ref[0] + p_ref[1]) * dinv
        hp = jax.nn.relu(
            oin_ref[...] + jnp.dot(hk, w_ref[...],
                                   preferred_element_type=jnp.float32,
                                   precision=_PREC) + b_ref[...])
        z = (jnp.dot(hp, wn_ref[...], preferred_element_type=jnp.float32,
                     precision=_PREC) + bn_ref[...])
        nrm = jnp.sqrt(jnp.sum(z * z, axis=-1, keepdims=True))
        out_ref[...] = z / jnp.maximum(nrm, 1e-12)

    return pl.pallas_call(
        body,
        grid=(GRID,),
        in_specs=[
            pl.BlockSpec((2, RB, fi), lambda i: (0, i, 0)),
            pl.BlockSpec((fi, fo), lambda i: (0, 0)),
            pl.BlockSpec((1, fo), lambda i: (0, 0)),
            pl.BlockSpec((2, RB), lambda i: (0, i)),
            pl.BlockSpec((RB, fo), lambda i: (i, 0)),
            pl.BlockSpec((fo, fon), lambda i: (0, 0)),
            pl.BlockSpec((1, fon), lambda i: (0, 0)),
        ],
        out_specs=pl.BlockSpec((RB, fon), lambda i: (i, 0)),
        out_shape=jax.ShapeDtypeStruct((N_PAD, fon), jnp.float32),
    )(P, W, b.reshape(1, -1), degp, out_in, fcW, fcb.reshape(1, -1))


def kernel(x, edge_index, edge_attr,
           W1_0, b1_0, W1_1, b1_1,
           W2_0, b2_0, W2_1, b2_1, W2_2, b2_2,
           W3_0, b3_0, W3_1, b3_1, W3_2, b3_2, W3_3, b3_3,
           fc_W, fc_b):
    npad = E_PAD - E
    pad_nodes = (jnp.arange(npad, dtype=jnp.int32) % (N_PAD - N)) + N
    row3 = jnp.concatenate([edge_index[0], pad_nodes]).reshape(
        NW, NCHUNK, CHUNK)
    col3 = jnp.concatenate([edge_index[1], pad_nodes]).reshape(
        NW, NCHUNK, CHUNK)
    w3 = jnp.concatenate(
        [edge_attr, jnp.zeros((npad,), jnp.float32)]).reshape(
        NW, NCHUNK, CHUNK)
    x_p = jnp.pad(x, ((0, N_PAD - N), (0, 0)))

    row2 = row3.reshape(-1, CHUNK)
    col2 = col3.reshape(-1, CHUNK)
    w2 = w3.reshape(-1, CHUNK)

    degp = _deg_kernel(col3, w3)

    out0, u = _tc_head(x_p, W1_0, b1_0, degp)
    P = _prop_ring(u, row2, col2, w2, 128, 2)
    out0, u = _tc_bridge(P, W1_1, b1_1, degp, out0, W2_0, b2_0)
    P = _prop(u, row3, col3, w3, 32)
    out0, u = _tc_mid(P, W2_1, b2_1, degp, out0)
    P = _prop(u, row3, col3, w3, 32)
    def prop64(u):
        return _prop(u, row3, col3, w3, 64)

    out0, u = _tc_bridge(P, W2_2, b2_2, degp, out0, W3_0, b3_0)
    P = prop64(u)
    out0, u = _tc_mid(P, W3_1, b3_1, degp, out0)
    P = prop64(u)
    out0, u = _tc_mid(P, W3_2, b3_2, degp, out0)
    P = prop64(u)
    final = _tc_tail(P, W3_3, b3_3, degp, out0, fc_W, fc_b)
    return final[:N]


# speedup vs baseline: 1.6015x; 1.4153x over previous
import functools

import jax
import jax.numpy as jnp
from jax import lax
from jax.experimental import pallas as pl
from jax.experimental.pallas import tpu as pltpu
from jax.experimental.pallas import tpu_sc as plsc

N = 10000
N_PAD = 10240
E = 320000
E_PAD = 327680
NC = 2
NS = 16
NW = NC * NS
EPT = E_PAD // NW
CHUNK = 128
NCHUNK = EPT // CHUNK
NBUF = 4
ROWS_PT = N_PAD // NS
RB = 1024
GRID = N_PAD // RB

_MESH = plsc.VectorSubcoreMesh(core_axis_name="c", subcore_axis_name="s")
_PREC = lax.Precision.HIGHEST
_SC_PARAMS = pltpu.CompilerParams(use_tc_tiling_on_sc=False)


def _deg_kernel(col3, w3):
    @functools.partial(
        pl.kernel,
        mesh=_MESH,
        compiler_params=_SC_PARAMS,
        out_type=jax.ShapeDtypeStruct((NC, N_PAD), jnp.float32),
        scratch_types=[
            pltpu.VMEM((NCHUNK, CHUNK), jnp.int32),
            pltpu.VMEM((NCHUNK, CHUNK), jnp.float32),
            pltpu.VMEM((CHUNK,), jnp.float32),
            pltpu.VMEM_SHARED((N_PAD,), jnp.float32),
            pltpu.SemaphoreType.DMA,
            pltpu.SemaphoreType.DMA,
        ],
    )
    def deg(col_hbm, w_hbm, out_hbm, cidx2, w2, zbuf_v, acc_sh, s0, s1):
        cid = lax.axis_index("c")
        sid = lax.axis_index("s")
        wid = sid * NC + cid
        ld0 = pltpu.async_copy(col_hbm.at[wid], cidx2, s0)
        ld1 = pltpu.async_copy(w_hbm.at[wid], w2, s1)
        for j in range(CHUNK // 16):
            zbuf_v[pl.ds(16 * j, 16)] = jnp.zeros((16,), jnp.float32)
        r0 = sid * ROWS_PT
        for k in range(ROWS_PT // CHUNK):
            pltpu.sync_copy(zbuf_v, acc_sh.at[pl.ds(r0 + k * CHUNK, CHUNK)])
        ld0.wait()
        ld1.wait()
        plsc.subcore_barrier()

        ssem = (s0, s1)

        def scat(i, b):
            return pltpu.async_copy(w2.at[i], acc_sh.at[cidx2.at[i]],
                                    ssem[b], add=True)

        def swait(i, b):
            pltpu.make_async_copy(w2.at[i], acc_sh.at[cidx2.at[i]],
                                  ssem[b]).wait()

        scat(0, 0)
        scat(1, 1)

        def body(g, carry):
            i0 = 2 * g
            swait(i0 - 2, 0)
            scat(i0, 0)
            swait(i0 - 1, 1)
            scat(i0 + 1, 1)
            return carry

        lax.fori_loop(1, NCHUNK // 2, body, 0)
        swait(NCHUNK - 2, 0)
        swait(NCHUNK - 1, 1)
        plsc.subcore_barrier()
        for k in range(ROWS_PT // CHUNK):
            off = r0 + k * CHUNK
            pltpu.sync_copy(acc_sh.at[pl.ds(off, CHUNK)], zbuf_v)
            pltpu.sync_copy(zbuf_v, out_hbm.at[cid, pl.ds(off, CHUNK)])

    return deg(col3, w3)


def _prop(u, row3, col3, w3, C):
    @functools.partial(
        pl.kernel,
        mesh=_MESH,
        compiler_params=_SC_PARAMS,
        out_type=jax.ShapeDtypeStruct((NC, N_PAD, C), jnp.float32),
        scratch_types=[
            pltpu.VMEM((NCHUNK, CHUNK), jnp.int32),
            pltpu.VMEM((NCHUNK, CHUNK), jnp.int32),
            pltpu.VMEM((NCHUNK, CHUNK), jnp.float32),
            pltpu.VMEM((CHUNK, C), jnp.float32),
            pltpu.VMEM((CHUNK, C), jnp.float32),
            pltpu.VMEM((CHUNK, C), jnp.float32),
            pltpu.VMEM((CHUNK, C), jnp.float32),
            pltpu.VMEM((CHUNK, C), jnp.float32),
            pltpu.VMEM_SHARED((N_PAD, C), jnp.float32),
            pltpu.SemaphoreType.DMA,
            pltpu.SemaphoreType.DMA,
            pltpu.SemaphoreType.DMA,
            pltpu.SemaphoreType.DMA,
            pltpu.SemaphoreType.DMA,
            pltpu.SemaphoreType.DMA,
            pltpu.SemaphoreType.DMA,
            pltpu.SemaphoreType.DMA,
        ],
    )
    def prop(u_hbm, row_hbm, col_hbm, w_hbm, out_hbm,
             ridx2, cidx2, w2, b0, b1, b2, b3, zbuf, acc_sh,
             g0, g1, g2, g3, s0, s1, s2, s3):
        bufs = (b0, b1, b2, b3)
        gsem = (g0, g1, g2, g3)
        ssem = (s0, s1, s2, s3)
        cid = lax.axis_index("c")
        sid = lax.axis_index("s")
        wid = sid * NC + cid

        ld = [pltpu.async_copy(row_hbm.at[wid], ridx2, g0),
              pltpu.async_copy(col_hbm.at[wid], cidx2, g1),
              pltpu.async_copy(w_hbm.at[wid], w2, g2)]

        def zbody(e, carry):
            for j in range(C // 16):
                zbuf[e, pl.ds(16 * j, 16)] = jnp.zeros((16,), jnp.float32)
            return carry

        lax.fori_loop(0, CHUNK, zbody, 0)
        r0 = sid * ROWS_PT
        for k in range(ROWS_PT // CHUNK):
            pltpu.sync_copy(zbuf, acc_sh.at[pl.ds(r0 + k * CHUNK, CHUNK)])
        for h in ld:
            h.wait()
        plsc.subcore_barrier()

        def gather(i, b):
            pltpu.async_copy(u_hbm.at[ridx2.at[i]], bufs[b], gsem[b])

        def gwait(i, b):
            pltpu.make_async_copy(u_hbm.at[ridx2.at[i]], bufs[b],
                                  gsem[b]).wait()

        def scat(i, b):
            pltpu.async_copy(bufs[b], acc_sh.at[cidx2.at[i]], ssem[b],
                             add=True)

        def swait(i, b):
            pltpu.make_async_copy(bufs[b], acc_sh.at[cidx2.at[i]],
                                  ssem[b]).wait()

        def mul(i, b):
            buf = bufs[b]

            def mbody(g, carry):
                wv = w2[i, pl.ds(16 * g, 16)]
                for k in range(16):
                    ws = wv[k]
                    for j in range(C // 16):
                        sl = pl.ds(16 * j, 16)
                        buf[16 * g + k, sl] = buf[16 * g + k, sl] * ws
                return carry

            lax.fori_loop(0, CHUNK // 16, mbody, 0)

        for j in range(3):
            gather(j, j)
        for i in range(NBUF):
            gwait(i, i)
            mul(i, i)
            scat(i, i)
            if i >= 1:
                swait(i - 1, i - 1)
            gather(i + 3, (i + 3) % NBUF)

        def body(g, carry):
            for b in range(NBUF):
                i = NBUF * g + b
                gwait(i, b)
                mul(i, b)
                scat(i, b)
                pb = (b + 3) % NBUF
                swait(i - 1, pb)
                i3 = i + 3

                @pl.when(i3 < NCHUNK)
                def _():
                    gather(i3, pb)

            return carry

        lax.fori_loop(1, NCHUNK // NBUF, body, 0)
        swait(NCHUNK - 1, (NCHUNK - 1) % NBUF)
        plsc.subcore_barrier()

        def acc_sl(k):
            return acc_sh.at[pl.ds(r0 + k * CHUNK, CHUNK)]

        def out_sl(k):
            return out_hbm.at[cid, pl.ds(r0 + k * CHUNK, CHUNK)]

        outs = []
        for k in range(5):
            b = k % 4
            if k >= 4:
                outs[b].wait()
            pltpu.sync_copy(acc_sl(k), bufs[b])
            outs.append(pltpu.async_copy(bufs[b], out_sl(k), ssem[b]))
        for k in range(1, 5):
            outs[k].wait()

    return prop(u, row3, col3, w3)


def _prop_ring(u, row2, col2, w2, C, nbuf, cmul=None):
    U = 8
    D = nbuf - 1
    if cmul is None:
        cmul = C

    @functools.partial(
        pl.kernel,
        mesh=_MESH,
        compiler_params=_SC_PARAMS,
        out_type=jax.ShapeDtypeStruct((NC, N_PAD, C), jnp.float32),
        scratch_types=[
            [pltpu.VMEM((CHUNK,), jnp.int32) for _ in range(U)],
            [pltpu.VMEM((CHUNK,), jnp.int32) for _ in range(U)],
            [pltpu.VMEM((CHUNK,), jnp.float32) for _ in range(U)],
            [pltpu.VMEM((CHUNK, C), jnp.float32) for _ in range(nbuf)],
            pltpu.VMEM_SHARED((N_PAD, C), jnp.float32),
            [pltpu.SemaphoreType.DMA for _ in range(U)],
            [pltpu.SemaphoreType.DMA for _ in range(nbuf)],
            [pltpu.SemaphoreType.DMA for _ in range(nbuf)],
        ],
    )
    def prop(u_hbm, row_hbm, col_hbm, w_hbm, out_hbm,
             ridx, cidx, wv8, bufs, acc_sh, isem, gsem, ssem):
        cid = lax.axis_index("c")
        sid = lax.axis_index("s")
        wid = sid * NC + cid
        base = wid * NCHUNK

        def iload(i, s):
            pltpu.async_copy(row_hbm.at[base + i], ridx[s], isem[s])
            pltpu.async_copy(col_hbm.at[base + i], cidx[s], isem[s])
            pltpu.async_copy(w_hbm.at[base + i], wv8[s], isem[s])

        def iwait(i, s):
            pltpu.make_async_copy(row_hbm.at[base + i], ridx[s],
                                  isem[s]).wait()
            pltpu.make_async_copy(col_hbm.at[base + i], cidx[s],
                                  isem[s]).wait()
            pltpu.make_async_copy(w_hbm.at[base + i], wv8[s], isem[s]).wait()

        def gather(i, b, s):
            pltpu.async_copy(u_hbm.at[ridx[s]], bufs[b], gsem[b])

        def gwait(i, b, s):
            pltpu.make_async_copy(u_hbm.at[ridx[s]], bufs[b], gsem[b]).wait()

        def scat(i, b, s):
            pltpu.async_copy(bufs[b], acc_sh.at[cidx[s]], ssem[b], add=True)

        def swait(i, b, s):
            pltpu.make_async_copy(bufs[b], acc_sh.at[cidx[s]],
                                  ssem[b]).wait()

        def mul(i, b, s):
            buf = bufs[b]

            def mbody(g, carry):
                wvec = wv8[s][pl.ds(16 * g, 16)]
                for k in range(16):
                    ws = wvec[k]
                    for j in range(cmul // 16):
                        sl = pl.ds(16 * j, 16)
                        buf[16 * g + k, sl] = buf[16 * g + k, sl] * ws
                return carry

            lax.fori_loop(0, CHUNK // 16, mbody, 0)

        def zbody(e, carry):
            for j in range(C // 16):
                bufs[0][e, pl.ds(16 * j, 16)] = jnp.zeros((16,), jnp.float32)
            return carry

        lax.fori_loop(0, CHUNK, zbody, 0)
        r0 = sid * ROWS_PT
        for k in range(ROWS_PT // CHUNK):
            pltpu.sync_copy(bufs[0], acc_sh.at[pl.ds(r0 + k * CHUNK, CHUNK)])
        plsc.subcore_barrier()

        def step(i, bs, static):
            bb = bs % nbuf
            gwait(i, bb, bs)
            if (not static) or bs >= 1:
                swait(i - 1, (bs - 1) % nbuf, (bs - 1) % U)
            nd = i + D
            nd_b, nd_s = (bs + D) % nbuf, (bs + D) % U
            ni = i + D + 2
            ni_s = (bs + D + 2) % U

            def do_gather():
                iwait(nd, nd_s)
                gather(nd, nd_b, nd_s)

            def do_iload():
                iload(ni, ni_s)

            if static:
                if nd < NCHUNK:
                    do_gather()
                if ni < NCHUNK:
                    do_iload()
            else:
                @pl.when(nd < NCHUNK)
                def _():
                    do_gather()

                @pl.when(ni < NCHUNK)
                def _():
                    do_iload()

            mul(i, bb, bs)
            scat(i, bb, bs)

        for j in range(D + 2):
            iload(j, j % U)
        for j in range(D):
            iwait(j, j % U)
            gather(j, j % nbuf, j % U)
        for i in range(U):
            step(i, i, static=True)

        def body(g, carry):
            for b in range(U):
                step(U * g + b, b, static=False)
            return carry

        lax.fori_loop(1, NCHUNK // U, body, 0)
        swait(NCHUNK - 1, (NCHUNK - 1) % nbuf, (NCHUNK - 1) % U)
        plsc.subcore_barrier()

        def acc_sl(k):
            return acc_sh.at[pl.ds(r0 + k * CHUNK, CHUNK)]

        def out_sl(k):
            return out_hbm.at[cid, pl.ds(r0 + k * CHUNK, CHUNK)]

        outs = []
        for k in range(5):
            b = k % nbuf
            if k >= nbuf:
                outs[k - nbuf].wait()
            pltpu.sync_copy(acc_sl(k), bufs[b])
            outs.append(pltpu.async_copy(bufs[b], out_sl(k), ssem[b]))
        for k in range(max(0, 5 - nbuf), 5):
            outs[k].wait()

    return prop(u, row2, col2, w2)


def _prop_dual(u0, u1, row2, col2, w2, nbuf=4):
    C = 32
    U = 8
    D = nbuf - 1

    @functools.partial(
        pl.kernel,
        mesh=_MESH,
        compiler_params=_SC_PARAMS,
        out_type=jax.ShapeDtypeStruct((NC, 2, N_PAD, C), jnp.float32),
        scratch_types=[
            [pltpu.VMEM((CHUNK,), jnp.int32) for _ in range(U)],
            [pltpu.VMEM((CHUNK,), jnp.int32) for _ in range(U)],
            [pltpu.VMEM((CHUNK,), jnp.float32) for _ in range(U)],
            [pltpu.VMEM((CHUNK, C), jnp.float32) for _ in range(2 * nbuf)],
            [pltpu.VMEM_SHARED((N_PAD, C), jnp.float32) for _ in range(2)],
            [pltpu.SemaphoreType.DMA for _ in range(U)],
            [pltpu.SemaphoreType.DMA for _ in range(2 * nbuf)],
            [pltpu.SemaphoreType.DMA for _ in range(2 * nbuf)],
        ],
    )
    def prop(u0_hbm, u1_hbm, row_hbm, col_hbm, w_hbm, out_hbm,
             ridx, cidx, wv8, bufs, accs, isem, gsem, ssem):
        u_hbm = (u0_hbm, u1_hbm)
        cid = lax.axis_index("c")
        sid = lax.axis_index("s")
        wid = sid * NC + cid
        base = wid * NCHUNK

        def iload(i, s):
            pltpu.async_copy(row_hbm.at[base + i], ridx[s], isem[s])
            pltpu.async_copy(col_hbm.at[base + i], cidx[s], isem[s])
            pltpu.async_copy(w_hbm.at[base + i], wv8[s], isem[s])

        def iwait(i, s):
            pltpu.make_async_copy(row_hbm.at[base + i], ridx[s],
                                  isem[s]).wait()
            pltpu.make_async_copy(col_hbm.at[base + i], cidx[s],
                                  isem[s]).wait()
            pltpu.make_async_copy(w_hbm.at[base + i], wv8[s], isem[s]).wait()

        def gather(i, b, s):
            for h in range(2):
                pltpu.async_copy(u_hbm[h].at[ridx[s]], bufs[2 * b + h],
                                 gsem[2 * b + h])

        def gwait(i, b, s):
            for h in range(2):
                pltpu.make_async_copy(u_hbm[h].at[ridx[s]], bufs[2 * b + h],
                                      gsem[2 * b + h]).wait()

        def scat(i, b, s):
            for h in range(2):
                pltpu.async_copy(bufs[2 * b + h], accs[h].at[cidx[s]],
                                 ssem[2 * b + h], add=True)

        def swait(i, b, s):
            for h in range(2):
                pltpu.make_async_copy(bufs[2 * b + h], accs[h].at[cidx[s]],
                                      ssem[2 * b + h]).wait()

        def mul(i, b, s):
            def mbody(g, carry):
                wvec = wv8[s][pl.ds(16 * g, 16)]
                for k in range(16):
                    ws = wvec[k]
                    for h in range(2):
                        buf = bufs[2 * b + h]
                        for j in range(C // 16):
                            sl = pl.ds(16 * j, 16)
                            buf[16 * g + k, sl] = buf[16 * g + k, sl] * ws
                return carry

            lax.fori_loop(0, CHUNK // 16, mbody, 0)

        def zbody(e, carry):
            for h in range(2):
                for j in range(C // 16):
                    bufs[h][e, pl.ds(16 * j, 16)] = jnp.zeros((16,),
                                                              jnp.float32)
            return carry

        lax.fori_loop(0, CHUNK, zbody, 0)
        r0 = sid * ROWS_PT
        for k in range(ROWS_PT // CHUNK):
            for h in range(2):
                pltpu.sync_copy(bufs[h],
                                accs[h].at[pl.ds(r0 + k * CHUNK, CHUNK)])
        plsc.subcore_barrier()

        def step(i, bs, static):
            bb = bs % nbuf
            gwait(i, bb, bs)
            if (not static) or bs >= 1:
                swait(i - 1, (bs - 1) % nbuf, (bs - 1) % U)
            nd = i + D
            nd_b, nd_s = (bs + D) % nbuf, (bs + D) % U
            ni = i + D + 2
            ni_s = (bs + D + 2) % U

            def do_gather():
                iwait(nd, nd_s)
                gather(nd, nd_b, nd_s)

            def do_iload():
                iload(ni, ni_s)

            if static:
                if nd < NCHUNK:
                    do_gather()
                if ni < NCHUNK:
                    do_iload()
            else:
                @pl.when(nd < NCHUNK)
                def _():
                    do_gather()

                @pl.when(ni < NCHUNK)
                def _():
                    do_iload()

            mul(i, bb, bs)
            scat(i, bb, bs)

        for j in range(D + 2):
            iload(j, j % U)
        for j in range(D):
            iwait(j, j % U)
            gather(j, j % nbuf, j % U)
        for i in range(U):
            step(i, i, static=True)

        def body(g, carry):
            for b in range(U):
                step(U * g + b, b, static=False)
            return carry

        lax.fori_loop(1, NCHUNK // U, body, 0)
        swait(NCHUNK - 1, (NCHUNK - 1) % nbuf, (NCHUNK - 1) % U)
        plsc.subcore_barrier()

        for h in range(2):
            outs = []
            for k in range(5):
                b = 2 * (k % nbuf) + h
                if k >= nbuf:
                    outs[k - nbuf].wait()
                pltpu.sync_copy(accs[h].at[pl.ds(r0 + k * CHUNK, CHUNK)],
                                bufs[b])
                outs.append(pltpu.async_copy(
                    bufs[b], out_hbm.at[cid, h, pl.ds(r0 + k * CHUNK, CHUNK)],
                    ssem[b]))
            for k in range(max(0, 5 - nbuf), 5):
                outs[k].wait()

    return prop(u0, u1, row2, col2, w2)


def _dinv_block(degp):
    deg = degp[0] + degp[1]
    safe = jnp.where(deg > 0, deg, 1.0)
    return jnp.where(deg > 0, lax.rsqrt(safe), 0.0).reshape(-1, 1)


def _tc_head(h, W, b, degp):
    fi, fo = W.shape

    def body(h_ref, w_ref, b_ref, degp_ref, out_ref, u_ref):
        dinv = _dinv_block(degp_ref[...])
        hv = h_ref[...]
        out_ref[...] = (
            jnp.dot(hv, w_ref[...], preferred_element_type=jnp.float32,
                    precision=_PREC) + b_ref[...])
        u_ref[...] = hv * dinv

    return pl.pallas_call(
        body,
        grid=(GRID,),
        in_specs=[
            pl.BlockSpec((RB, fi), lambda i: (i, 0)),
            pl.BlockSpec((fi, fo), lambda i: (0, 0)),
            pl.BlockSpec((1, fo), lambda i: (0, 0)),
            pl.BlockSpec((2, RB), lambda i: (0, i)),
        ],
        out_specs=[
            pl.BlockSpec((RB, fo), lambda i: (i, 0)),
            pl.BlockSpec((RB, fi), lambda i: (i, 0)),
        ],
        out_shape=[
            jax.ShapeDtypeStruct((N_PAD, fo), jnp.float32),
            jax.ShapeDtypeStruct((N_PAD, fi), jnp.float32),
        ],
    )(h, W, b.reshape(1, -1), degp)


def _tc_mid(P, W, b, degp, out_in):
    fi, fo = W.shape

    def body(p_ref, w_ref, b_ref, degp_ref, oin_ref, out_ref, u_ref):
        dinv = _dinv_block(degp_ref[...])
        hk = (p_ref[0] + p_ref[1]) * dinv
        out_ref[...] = oin_ref[...] + (
            jnp.dot(hk, w_ref[...], preferred_element_type=jnp.float32,
                    precision=_PREC) + b_ref[...])
        u_ref[...] = hk * dinv

    return pl.pallas_call(
        body,
        grid=(GRID,),
        in_specs=[
            pl.BlockSpec((2, RB, fi), lambda i: (0, i, 0)),
            pl.BlockSpec((fi, fo), lambda i: (0, 0)),
            pl.BlockSpec((1, fo), lambda i: (0, 0)),
            pl.BlockSpec((2, RB), lambda i: (0, i)),
            pl.BlockSpec((RB, fo), lambda i: (i, 0)),
        ],
        out_specs=[
            pl.BlockSpec((RB, fo), lambda i: (i, 0)),
            pl.BlockSpec((RB, fi), lambda i: (i, 0)),
        ],
        out_shape=[
            jax.ShapeDtypeStruct((N_PAD, fo), jnp.float32),
            jax.ShapeDtypeStruct((N_PAD, fi), jnp.float32),
        ],
    )(P, W, b.reshape(1, -1), degp, out_in)


def _tc_bridge(P, W, b, degp, out_in, Wn, bn):
    fi, fo = W.shape
    fon = Wn.shape[1]

    def body(p_ref, w_ref, b_ref, degp_ref, oin_ref, wn_ref, bn_ref,
             out_ref, u_ref):
        dinv = _dinv_block(degp_ref[...])
        hk = (p_ref[0] + p_ref[1]) * dinv
        hp = jax.nn.relu(
            oin_ref[...] + jnp.dot(hk, w_ref[...],
                                   preferred_element_type=jnp.float32,
                                   precision=_PREC) + b_ref[...])
        out_ref[...] = (
            jnp.dot(hp, wn_ref[...], preferred_element_type=jnp.float32,
                    precision=_PREC) + bn_ref[...])
        u_ref[...] = hp * dinv

    return pl.pallas_call(
        body,
        grid=(GRID,),
        in_specs=[
            pl.BlockSpec((2, RB, fi), lambda i: (0, i, 0)),
            pl.BlockSpec((fi, fo), lambda i: (0, 0)),
            pl.BlockSpec((1, fo), lambda i: (0, 0)),
            pl.BlockSpec((2, RB), lambda i: (0, i)),
            pl.BlockSpec((RB, fo), lambda i: (i, 0)),
            pl.BlockSpec((fo, fon), lambda i: (0, 0)),
            pl.BlockSpec((1, fon), lambda i: (0, 0)),
        ],
        out_specs=[
            pl.BlockSpec((RB, fon), lambda i: (i, 0)),
            pl.BlockSpec((RB, fo), lambda i: (i, 0)),
        ],
        out_shape=[
            jax.ShapeDtypeStruct((N_PAD, fon), jnp.float32),
            jax.ShapeDtypeStruct((N_PAD, fo), jnp.float32),
        ],
    )(P, W, b.reshape(1, -1), degp, out_in, Wn, bn.reshape(1, -1))


def _tc_bridge_d(P, W, b, degp, out_in, Wn, bn):
    fi, fo = W.shape
    fon = Wn.shape[1]

    def body(p_ref, w_ref, b_ref, degp_ref, oin_ref, wn_ref, bn_ref,
             out_ref, u0_ref, u1_ref):
        dinv = _dinv_block(degp_ref[...])
        hk = (p_ref[0] + p_ref[1]) * dinv
        hp = jax.nn.relu(
            oin_ref[...] + jnp.dot(hk, w_ref[...],
                                   preferred_element_type=jnp.float32,
                                   precision=_PREC) + b_ref[...])
        out_ref[...] = (
            jnp.dot(hp, wn_ref[...], preferred_element_type=jnp.float32,
                    precision=_PREC) + bn_ref[...])
        us = hp * dinv
        u0_ref[...] = us[:, :fo // 2]
        u1_ref[...] = us[:, fo // 2:]

    return pl.pallas_call(
        body,
        grid=(GRID,),
        in_specs=[
            pl.BlockSpec((2, RB, fi), lambda i: (0, i, 0)),
            pl.BlockSpec((fi, fo), lambda i: (0, 0)),
            pl.BlockSpec((1, fo), lambda i: (0, 0)),
            pl.BlockSpec((2, RB), lambda i: (0, i)),
            pl.BlockSpec((RB, fo), lambda i: (i, 0)),
            pl.BlockSpec((fo, fon), lambda i: (0, 0)),
            pl.BlockSpec((1, fon), lambda i: (0, 0)),
        ],
        out_specs=[
            pl.BlockSpec((RB, fon), lambda i: (i, 0)),
            pl.BlockSpec((RB, fo // 2), lambda i: (i, 0)),
            pl.BlockSpec((RB, fo // 2), lambda i: (i, 0)),
        ],
        out_shape=[
            jax.ShapeDtypeStruct((N_PAD, fon), jnp.float32),
            jax.ShapeDtypeStruct((N_PAD, fo // 2), jnp.float32),
            jax.ShapeDtypeStruct((N_PAD, fo // 2), jnp.float32),
        ],
    )(P, W, b.reshape(1, -1), degp, out_in, Wn, bn.reshape(1, -1))


def _tc_mid_d(Pd, W, b, degp, out_in):
    fi, fo = W.shape
    ch = fi // 2

    def body(p_ref, w_ref, b_ref, degp_ref, oin_ref,
             out_ref, u0_ref, u1_ref):
        dinv = _dinv_block(degp_ref[...])
        s = p_ref[0] + p_ref[1]
        hk = jnp.concatenate([s[0], s[1]], axis=-1) * dinv
        out_ref[...] = oin_ref[...] + (
            jnp.dot(hk, w_ref[...], preferred_element_type=jnp.float32,
                    precision=_PREC) + b_ref[...])
        us = hk * dinv
        u0_ref[...] = us[:, :ch]
        u1_ref[...] = us[:, ch:]

    return pl.pallas_call(
        body,
        grid=(GRID,),
        in_specs=[
            pl.BlockSpec((2, 2, RB, ch), lambda i: (0, 0, i, 0)),
            pl.BlockSpec((fi, fo), lambda i: (0, 0)),
            pl.BlockSpec((1, fo), lambda i: (0, 0)),
            pl.BlockSpec((2, RB), lambda i: (0, i)),
            pl.BlockSpec((RB, fo), lambda i: (i, 0)),
        ],
        out_specs=[
            pl.BlockSpec((RB, fo), lambda i: (i, 0)),
            pl.BlockSpec((RB, ch), lambda i: (i, 0)),
            pl.BlockSpec((RB, ch), lambda i: (i, 0)),
        ],
        out_shape=[
            jax.ShapeDtypeStruct((N_PAD, fo), jnp.float32),
            jax.ShapeDtypeStruct((N_PAD, ch), jnp.float32),
            jax.ShapeDtypeStruct((N_PAD, ch), jnp.float32),
        ],
    )(Pd, W, b.reshape(1, -1), degp, out_in)


def _tc_tail_d(Pd, W, b, degp, out_in, fcW, fcb):
    fi, fo = W.shape
    ch = fi // 2
    fon = fcW.shape[1]

    def body(p_ref, w_ref, b_ref, degp_ref, oin_ref, wn_ref, bn_ref,
             out_ref):
        dinv = _dinv_block(degp_ref[...])
        s = p_ref[0] + p_ref[1]
        hk = jnp.concatenate([s[0], s[1]], axis=-1) * dinv
        hp = jax.nn.relu(
            oin_ref[...] + jnp.dot(hk, w_ref[...],
                                   preferred_element_type=jnp.float32,
                                   precision=_PREC) + b_ref[...])
        z = (jnp.dot(hp, wn_ref[...], preferred_element_type=jnp.float32,
                     precision=_PREC) + bn_ref[...])
        nrm = jnp.sqrt(jnp.sum(z * z, axis=-1, keepdims=True))
        out_ref[...] = z / jnp.maximum(nrm, 1e-12)

    return pl.pallas_call(
        body,
        grid=(GRID,),
        in_specs=[
            pl.BlockSpec((2, 2, RB, ch), lambda i: (0, 0, i, 0)),
            pl.BlockSpec((fi, fo), lambda i: (0, 0)),
            pl.BlockSpec((1, fo), lambda i: (0, 0)),
            pl.BlockSpec((2, RB), lambda i: (0, i)),
            pl.BlockSpec((RB, fo), lambda i: (i, 0)),
            pl.BlockSpec((fo, fon), lambda i: (0, 0)),
            pl.BlockSpec((1, fon), lambda i: (0, 0)),
        ],
        out_specs=pl.BlockSpec((RB, fon), lambda i: (i, 0)),
        out_shape=jax.ShapeDtypeStruct((N_PAD, fon), jnp.float32),
    )(Pd, W, b.reshape(1, -1), degp, out_in, fcW, fcb.reshape(1, -1))


def _tc_tail(P, W, b, degp, out_in, fcW, fcb):
    fi, fo = W.shape
    fon = fcW.shape[1]

    def body(p_ref, w_ref, b_ref, degp_ref, oin_ref, wn_ref, bn_ref, out_ref):
        dinv = _dinv_block(degp_ref[...])
        hk = (p_ref[0] + p_ref[1]) * dinv
        hp = jax.nn.relu(
            oin_ref[...] + jnp.dot(hk, w_ref[...],
                                   preferred_element_type=jnp.float32,
                                   precision=_PREC) + b_ref[...])
        z = (jnp.dot(hp, wn_ref[...], preferred_element_type=jnp.float32,
                     precision=_PREC) + bn_ref[...])
        nrm = jnp.sqrt(jnp.sum(z * z, axis=-1, keepdims=True))
        out_ref[...] = z / jnp.maximum(nrm, 1e-12)

    return pl.pallas_call(
        body,
        grid=(GRID,),
        in_specs=[
            pl.BlockSpec((2, RB, fi), lambda i: (0, i, 0)),
            pl.BlockSpec((fi, fo), lambda i: (0, 0)),
            pl.BlockSpec((1, fo), lambda i: (0, 0)),
            pl.BlockSpec((2, RB), lambda i: (0, i)),
            pl.BlockSpec((RB, fo), lambda i: (i, 0)),
            pl.BlockSpec((fo, fon), lambda i: (0, 0)),
            pl.BlockSpec((1, fon), lambda i: (0, 0)),
        ],
        out_specs=pl.BlockSpec((RB, fon), lambda i: (i, 0)),
        out_shape=jax.ShapeDtypeStruct((N_PAD, fon), jnp.float32),
    )(P, W, b.reshape(1, -1), degp, out_in, fcW, fcb.reshape(1, -1))


def kernel(x, edge_index, edge_attr,
           W1_0, b1_0, W1_1, b1_1,
           W2_0, b2_0, W2_1, b2_1, W2_2, b2_2,
           W3_0, b3_0, W3_1, b3_1, W3_2, b3_2, W3_3, b3_3,
           fc_W, fc_b):
    npad = E_PAD - E
    pad_nodes = (jnp.arange(npad, dtype=jnp.int32) % (N_PAD - N)) + N
    row3 = jnp.concatenate([edge_index[0], pad_nodes]).reshape(
        NW, NCHUNK, CHUNK)
    col3 = jnp.concatenate([edge_index[1], pad_nodes]).reshape(
        NW, NCHUNK, CHUNK)
    w3 = jnp.concatenate(
        [edge_attr, jnp.zeros((npad,), jnp.float32)]).reshape(
        NW, NCHUNK, CHUNK)
    x_p = jnp.pad(x, ((0, N_PAD - N), (0, 0)))

    row2 = row3.reshape(-1, CHUNK)
    col2 = col3.reshape(-1, CHUNK)
    w2 = w3.reshape(-1, CHUNK)

    degp = _deg_kernel(col3, w3)

    out0, u = _tc_head(x_p, W1_0, b1_0, degp)
    P = _prop_ring(u, row2, col2, w2, 128, 2)
    out0, u = _tc_bridge(P, W1_1, b1_1, degp, out0, W2_0, b2_0)
    P = _prop(u, row3, col3, w3, 32)
    out0, u = _tc_mid(P, W2_1, b2_1, degp, out0)
    P = _prop(u, row3, col3, w3, 32)
    out0, u0, u1 = _tc_bridge_d(P, W2_2, b2_2, degp, out0, W3_0, b3_0)
    Pd = _prop_dual(u0, u1, row2, col2, w2)
    out0, u0, u1 = _tc_mid_d(Pd, W3_1, b3_1, degp, out0)
    Pd = _prop_dual(u0, u1, row2, col2, w2)
    out0, u0, u1 = _tc_mid_d(Pd, W3_2, b3_2, degp, out0)
    Pd = _prop_dual(u0, u1, row2, col2, w2)
    final = _tc_tail_d(Pd, W3_3, b3_3, degp, out0, fc_W, fc_b)
    return final[:N]


# speedup vs baseline: 1.6150x; 1.0084x over previous
import functools

import jax
import jax.numpy as jnp
from jax import lax
from jax.experimental import pallas as pl
from jax.experimental.pallas import tpu as pltpu
from jax.experimental.pallas import tpu_sc as plsc

N = 10000
N_PAD = 10240
E = 320000
E_PAD = 327680
NC = 2
NS = 16
NW = NC * NS
EPT = E_PAD // NW
CHUNK = 128
NCHUNK = EPT // CHUNK
NBUF = 4
ROWS_PT = N_PAD // NS
RB = 1024
GRID = N_PAD // RB

_MESH = plsc.VectorSubcoreMesh(core_axis_name="c", subcore_axis_name="s")
_PREC = lax.Precision.HIGHEST
_SC_PARAMS = pltpu.CompilerParams(use_tc_tiling_on_sc=False)


def _deg_kernel(col3, w3):
    @functools.partial(
        pl.kernel,
        mesh=_MESH,
        compiler_params=_SC_PARAMS,
        out_type=jax.ShapeDtypeStruct((NC, N_PAD), jnp.float32),
        scratch_types=[
            pltpu.VMEM((NCHUNK, CHUNK), jnp.int32),
            pltpu.VMEM((NCHUNK, CHUNK), jnp.float32),
            pltpu.VMEM((CHUNK,), jnp.float32),
            pltpu.VMEM_SHARED((N_PAD,), jnp.float32),
            pltpu.SemaphoreType.DMA,
            pltpu.SemaphoreType.DMA,
        ],
    )
    def deg(col_hbm, w_hbm, out_hbm, cidx2, w2, zbuf_v, acc_sh, s0, s1):
        cid = lax.axis_index("c")
        sid = lax.axis_index("s")
        wid = sid * NC + cid
        ld0 = pltpu.async_copy(col_hbm.at[wid], cidx2, s0)
        ld1 = pltpu.async_copy(w_hbm.at[wid], w2, s1)
        for j in range(CHUNK // 16):
            zbuf_v[pl.ds(16 * j, 16)] = jnp.zeros((16,), jnp.float32)
        r0 = sid * ROWS_PT
        for k in range(ROWS_PT // CHUNK):
            pltpu.sync_copy(zbuf_v, acc_sh.at[pl.ds(r0 + k * CHUNK, CHUNK)])
        ld0.wait()
        ld1.wait()
        plsc.subcore_barrier()

        ssem = (s0, s1)

        def scat(i, b):
            return pltpu.async_copy(w2.at[i], acc_sh.at[cidx2.at[i]],
                                    ssem[b], add=True)

        def swait(i, b):
            pltpu.make_async_copy(w2.at[i], acc_sh.at[cidx2.at[i]],
                                  ssem[b]).wait()

        scat(0, 0)
        scat(1, 1)

        def body(g, carry):
            i0 = 2 * g
            swait(i0 - 2, 0)
            scat(i0, 0)
            swait(i0 - 1, 1)
            scat(i0 + 1, 1)
            return carry

        lax.fori_loop(1, NCHUNK // 2, body, 0)
        swait(NCHUNK - 2, 0)
        swait(NCHUNK - 1, 1)
        plsc.subcore_barrier()
        for k in range(ROWS_PT // CHUNK):
            off = r0 + k * CHUNK
            pltpu.sync_copy(acc_sh.at[pl.ds(off, CHUNK)], zbuf_v)
            pltpu.sync_copy(zbuf_v, out_hbm.at[cid, pl.ds(off, CHUNK)])

    return deg(col3, w3)


def _prop(u, row3, col3, w3, C):
    @functools.partial(
        pl.kernel,
        mesh=_MESH,
        compiler_params=_SC_PARAMS,
        out_type=jax.ShapeDtypeStruct((NC, N_PAD, C), jnp.float32),
        scratch_types=[
            pltpu.VMEM((NCHUNK, CHUNK), jnp.int32),
            pltpu.VMEM((NCHUNK, CHUNK), jnp.int32),
            pltpu.VMEM((NCHUNK, CHUNK), jnp.float32),
            pltpu.VMEM((CHUNK, C), jnp.float32),
            pltpu.VMEM((CHUNK, C), jnp.float32),
            pltpu.VMEM((CHUNK, C), jnp.float32),
            pltpu.VMEM((CHUNK, C), jnp.float32),
            pltpu.VMEM((CHUNK, C), jnp.float32),
            pltpu.VMEM_SHARED((N_PAD, C), jnp.float32),
            pltpu.SemaphoreType.DMA,
            pltpu.SemaphoreType.DMA,
            pltpu.SemaphoreType.DMA,
            pltpu.SemaphoreType.DMA,
            pltpu.SemaphoreType.DMA,
            pltpu.SemaphoreType.DMA,
            pltpu.SemaphoreType.DMA,
            pltpu.SemaphoreType.DMA,
        ],
    )
    def prop(u_hbm, row_hbm, col_hbm, w_hbm, out_hbm,
             ridx2, cidx2, w2, b0, b1, b2, b3, zbuf, acc_sh,
             g0, g1, g2, g3, s0, s1, s2, s3):
        bufs = (b0, b1, b2, b3)
        gsem = (g0, g1, g2, g3)
        ssem = (s0, s1, s2, s3)
        cid = lax.axis_index("c")
        sid = lax.axis_index("s")
        wid = sid * NC + cid

        ld = [pltpu.async_copy(row_hbm.at[wid], ridx2, g0),
              pltpu.async_copy(col_hbm.at[wid], cidx2, g1),
              pltpu.async_copy(w_hbm.at[wid], w2, g2)]

        def zbody(e, carry):
            for j in range(C // 16):
                zbuf[e, pl.ds(16 * j, 16)] = jnp.zeros((16,), jnp.float32)
            return carry

        for h in ld:
            h.wait()
        for j in range(3):
            pltpu.async_copy(u_hbm.at[ridx2.at[j]], bufs[j], gsem[j])
        lax.fori_loop(0, CHUNK, zbody, 0)
        r0 = sid * ROWS_PT
        for k in range(ROWS_PT // CHUNK):
            pltpu.sync_copy(zbuf, acc_sh.at[pl.ds(r0 + k * CHUNK, CHUNK)])
        plsc.subcore_barrier()

        def gather(i, b):
            pltpu.async_copy(u_hbm.at[ridx2.at[i]], bufs[b], gsem[b])

        def gwait(i, b):
            pltpu.make_async_copy(u_hbm.at[ridx2.at[i]], bufs[b],
                                  gsem[b]).wait()

        def scat(i, b):
            pltpu.async_copy(bufs[b], acc_sh.at[cidx2.at[i]], ssem[b],
                             add=True)

        def swait(i, b):
            pltpu.make_async_copy(bufs[b], acc_sh.at[cidx2.at[i]],
                                  ssem[b]).wait()

        def mul(i, b):
            buf = bufs[b]

            def mbody(g, carry):
                wv = w2[i, pl.ds(16 * g, 16)]
                for k in range(16):
                    ws = wv[k]
                    for j in range(C // 16):
                        sl = pl.ds(16 * j, 16)
                        buf[16 * g + k, sl] = buf[16 * g + k, sl] * ws
                return carry

            lax.fori_loop(0, CHUNK // 16, mbody, 0)

        for i in range(NBUF):
            gwait(i, i)
            mul(i, i)
            scat(i, i)
            if i >= 1:
                swait(i - 1, i - 1)
            gather(i + 3, (i + 3) % NBUF)

        def body(g, carry):
            for b in range(NBUF):
                i = NBUF * g + b
                gwait(i, b)
                mul(i, b)
                scat(i, b)
                pb = (b + 3) % NBUF
                swait(i - 1, pb)
                i3 = i + 3

                @pl.when(i3 < NCHUNK)
                def _():
                    gather(i3, pb)

            return carry

        lax.fori_loop(1, NCHUNK // NBUF, body, 0)
        swait(NCHUNK - 1, (NCHUNK - 1) % NBUF)
        plsc.subcore_barrier()

        def acc_sl(k):
            return acc_sh.at[pl.ds(r0 + k * CHUNK, CHUNK)]

        def out_sl(k):
            return out_hbm.at[cid, pl.ds(r0 + k * CHUNK, CHUNK)]

        outs = []
        for k in range(5):
            b = k % 4
            if k >= 4:
                outs[b].wait()
            pltpu.sync_copy(acc_sl(k), bufs[b])
            outs.append(pltpu.async_copy(bufs[b], out_sl(k), ssem[b]))
        for k in range(1, 5):
            outs[k].wait()

    return prop(u, row3, col3, w3)


def _prop_ring(u, row2, col2, w2, C, nbuf, cmul=None):
    U = 8
    D = nbuf - 1
    if cmul is None:
        cmul = C

    @functools.partial(
        pl.kernel,
        mesh=_MESH,
        compiler_params=_SC_PARAMS,
        out_type=jax.ShapeDtypeStruct((NC, N_PAD, C), jnp.float32),
        scratch_types=[
            [pltpu.VMEM((CHUNK,), jnp.int32) for _ in range(U)],
            [pltpu.VMEM((CHUNK,), jnp.int32) for _ in range(U)],
            [pltpu.VMEM((CHUNK,), jnp.float32) for _ in range(U)],
            [pltpu.VMEM((CHUNK, C), jnp.float32) for _ in range(nbuf)],
            pltpu.VMEM_SHARED((N_PAD, C), jnp.float32),
            [pltpu.SemaphoreType.DMA for _ in range(U)],
            [pltpu.SemaphoreType.DMA for _ in range(nbuf)],
            [pltpu.SemaphoreType.DMA for _ in range(nbuf)],
        ],
    )
    def prop(u_hbm, row_hbm, col_hbm, w_hbm, out_hbm,
             ridx, cidx, wv8, bufs, acc_sh, isem, gsem, ssem):
        cid = lax.axis_index("c")
        sid = lax.axis_index("s")
        wid = sid * NC + cid
        base = wid * NCHUNK

        def iload(i, s):
            pltpu.async_copy(row_hbm.at[base + i], ridx[s], isem[s])
            pltpu.async_copy(col_hbm.at[base + i], cidx[s], isem[s])
            pltpu.async_copy(w_hbm.at[base + i], wv8[s], isem[s])

        def iwait(i, s):
            pltpu.make_async_copy(row_hbm.at[base + i], ridx[s],
                                  isem[s]).wait()
            pltpu.make_async_copy(col_hbm.at[base + i], cidx[s],
                                  isem[s]).wait()
            pltpu.make_async_copy(w_hbm.at[base + i], wv8[s], isem[s]).wait()

        def gather(i, b, s):
            pltpu.async_copy(u_hbm.at[ridx[s]], bufs[b], gsem[b])

        def gwait(i, b, s):
            pltpu.make_async_copy(u_hbm.at[ridx[s]], bufs[b], gsem[b]).wait()

        def scat(i, b, s):
            pltpu.async_copy(bufs[b], acc_sh.at[cidx[s]], ssem[b], add=True)

        def swait(i, b, s):
            pltpu.make_async_copy(bufs[b], acc_sh.at[cidx[s]],
                                  ssem[b]).wait()

        def mul(i, b, s):
            buf = bufs[b]

            def mbody(g, carry):
                wvec = wv8[s][pl.ds(16 * g, 16)]
                for k in range(16):
                    ws = wvec[k]
                    for j in range(cmul // 16):
                        sl = pl.ds(16 * j, 16)
                        buf[16 * g + k, sl] = buf[16 * g + k, sl] * ws
                return carry

            lax.fori_loop(0, CHUNK // 16, mbody, 0)

        for j in range(D + 2):
            iload(j, j % U)
        for j in range(D):
            iwait(j, j % U)
            gather(j, j % nbuf, j % U)
        zb = bufs[D % nbuf]

        def zbody(e, carry):
            for j in range(C // 16):
                zb[e, pl.ds(16 * j, 16)] = jnp.zeros((16,), jnp.float32)
            return carry

        lax.fori_loop(0, CHUNK, zbody, 0)
        r0 = sid * ROWS_PT
        for k in range(ROWS_PT // CHUNK):
            pltpu.sync_copy(zb, acc_sh.at[pl.ds(r0 + k * CHUNK, CHUNK)])
        plsc.subcore_barrier()

        def step(i, bs, static):
            bb = bs % nbuf
            gwait(i, bb, bs)
            if (not static) or bs >= 1:
                swait(i - 1, (bs - 1) % nbuf, (bs - 1) % U)
            nd = i + D
            nd_b, nd_s = (bs + D) % nbuf, (bs + D) % U
            ni = i + D + 2
            ni_s = (bs + D + 2) % U

            def do_gather():
                iwait(nd, nd_s)
                gather(nd, nd_b, nd_s)

            def do_iload():
                iload(ni, ni_s)

            if static:
                if nd < NCHUNK:
                    do_gather()
                if ni < NCHUNK:
                    do_iload()
            else:
                @pl.when(nd < NCHUNK)
                def _():
                    do_gather()

                @pl.when(ni < NCHUNK)
                def _():
                    do_iload()

            mul(i, bb, bs)
            scat(i, bb, bs)

        for i in range(U):
            step(i, i, static=True)

        def body(g, carry):
            for b in range(U):
                step(U * g + b, b, static=False)
            return carry

        lax.fori_loop(1, NCHUNK // U, body, 0)
        swait(NCHUNK - 1, (NCHUNK - 1) % nbuf, (NCHUNK - 1) % U)
        plsc.subcore_barrier()

        def acc_sl(k):
            return acc_sh.at[pl.ds(r0 + k * CHUNK, CHUNK)]

        def out_sl(k):
            return out_hbm.at[cid, pl.ds(r0 + k * CHUNK, CHUNK)]

        outs = []
        for k in range(5):
            b = k % nbuf
            if k >= nbuf:
                outs[k - nbuf].wait()
            pltpu.sync_copy(acc_sl(k), bufs[b])
            outs.append(pltpu.async_copy(bufs[b], out_sl(k), ssem[b]))
        for k in range(max(0, 5 - nbuf), 5):
            outs[k].wait()

    return prop(u, row2, col2, w2)


def _prop_dual(u0, u1, row2, col2, w2, nbuf=4):
    C = 32
    U = 8
    D = nbuf - 1

    @functools.partial(
        pl.kernel,
        mesh=_MESH,
        compiler_params=_SC_PARAMS,
        out_type=jax.ShapeDtypeStruct((NC, 2, N_PAD, C), jnp.float32),
        scratch_types=[
            [pltpu.VMEM((CHUNK,), jnp.int32) for _ in range(U)],
            [pltpu.VMEM((CHUNK,), jnp.int32) for _ in range(U)],
            [pltpu.VMEM((CHUNK,), jnp.float32) for _ in range(U)],
            [pltpu.VMEM((CHUNK, C), jnp.float32) for _ in range(2 * nbuf)],
            [pltpu.VMEM_SHARED((N_PAD, C), jnp.float32) for _ in range(2)],
            [pltpu.SemaphoreType.DMA for _ in range(U)],
            [pltpu.SemaphoreType.DMA for _ in range(2 * nbuf)],
            [pltpu.SemaphoreType.DMA for _ in range(2 * nbuf)],
        ],
    )
    def prop(u0_hbm, u1_hbm, row_hbm, col_hbm, w_hbm, out_hbm,
             ridx, cidx, wv8, bufs, accs, isem, gsem, ssem):
        u_hbm = (u0_hbm, u1_hbm)
        cid = lax.axis_index("c")
        sid = lax.axis_index("s")
        wid = sid * NC + cid
        base = wid * NCHUNK

        def iload(i, s):
            pltpu.async_copy(row_hbm.at[base + i], ridx[s], isem[s])
            pltpu.async_copy(col_hbm.at[base + i], cidx[s], isem[s])
            pltpu.async_copy(w_hbm.at[base + i], wv8[s], isem[s])

        def iwait(i, s):
            pltpu.make_async_copy(row_hbm.at[base + i], ridx[s],
                                  isem[s]).wait()
            pltpu.make_async_copy(col_hbm.at[base + i], cidx[s],
                                  isem[s]).wait()
            pltpu.make_async_copy(w_hbm.at[base + i], wv8[s], isem[s]).wait()

        def gather(i, b, s):
            for h in range(2):
                pltpu.async_copy(u_hbm[h].at[ridx[s]], bufs[2 * b + h],
                                 gsem[2 * b + h])

        def gwait(i, b, s):
            for h in range(2):
                pltpu.make_async_copy(u_hbm[h].at[ridx[s]], bufs[2 * b + h],
                                      gsem[2 * b + h]).wait()

        def scat(i, b, s):
            for h in range(2):
                pltpu.async_copy(bufs[2 * b + h], accs[h].at[cidx[s]],
                                 ssem[2 * b + h], add=True)

        def swait(i, b, s):
            for h in range(2):
                pltpu.make_async_copy(bufs[2 * b + h], accs[h].at[cidx[s]],
                                      ssem[2 * b + h]).wait()

        def mul(i, b, s):
            def mbody(g, carry):
                wvec = wv8[s][pl.ds(16 * g, 16)]
                for k in range(16):
                    ws = wvec[k]
                    for h in range(2):
                        buf = bufs[2 * b + h]
                        for j in range(C // 16):
                            sl = pl.ds(16 * j, 16)
                            buf[16 * g + k, sl] = buf[16 * g + k, sl] * ws
                return carry

            lax.fori_loop(0, CHUNK // 16, mbody, 0)

        for j in range(D + 2):
            iload(j, j % U)
        for j in range(D):
            iwait(j, j % U)
            gather(j, j % nbuf, j % U)

        def zbody(e, carry):
            for h in range(2):
                for j in range(C // 16):
                    bufs[2 * D + h][e, pl.ds(16 * j, 16)] = jnp.zeros(
                        (16,), jnp.float32)
            return carry

        lax.fori_loop(0, CHUNK, zbody, 0)
        r0 = sid * ROWS_PT
        for k in range(ROWS_PT // CHUNK):
            for h in range(2):
                pltpu.sync_copy(bufs[2 * D + h],
                                accs[h].at[pl.ds(r0 + k * CHUNK, CHUNK)])
        plsc.subcore_barrier()

        def step(i, bs, static):
            bb = bs % nbuf
            gwait(i, bb, bs)
            if (not static) or bs >= 1:
                swait(i - 1, (bs - 1) % nbuf, (bs - 1) % U)
            nd = i + D
            nd_b, nd_s = (bs + D) % nbuf, (bs + D) % U
            ni = i + D + 2
            ni_s = (bs + D + 2) % U

            def do_gather():
                iwait(nd, nd_s)
                gather(nd, nd_b, nd_s)

            def do_iload():
                iload(ni, ni_s)

            if static:
                if nd < NCHUNK:
                    do_gather()
                if ni < NCHUNK:
                    do_iload()
            else:
                @pl.when(nd < NCHUNK)
                def _():
                    do_gather()

                @pl.when(ni < NCHUNK)
                def _():
                    do_iload()

            mul(i, bb, bs)
            scat(i, bb, bs)

        for i in range(U):
            step(i, i, static=True)

        def body(g, carry):
            for b in range(U):
                step(U * g + b, b, static=False)
            return carry

        lax.fori_loop(1, NCHUNK // U, body, 0)
        swait(NCHUNK - 1, (NCHUNK - 1) % nbuf, (NCHUNK - 1) % U)
        plsc.subcore_barrier()

        for h in range(2):
            outs = []
            for k in range(5):
                b = 2 * (k % nbuf) + h
                if k >= nbuf:
                    outs[k - nbuf].wait()
                pltpu.sync_copy(accs[h].at[pl.ds(r0 + k * CHUNK, CHUNK)],
                                bufs[b])
                outs.append(pltpu.async_copy(
                    bufs[b], out_hbm.at[cid, h, pl.ds(r0 + k * CHUNK, CHUNK)],
                    ssem[b]))
            for k in range(max(0, 5 - nbuf), 5):
                outs[k].wait()

    return prop(u0, u1, row2, col2, w2)


def _dinv_block(degp):
    deg = degp[0] + degp[1]
    safe = jnp.where(deg > 0, deg, 1.0)
    return jnp.where(deg > 0, lax.rsqrt(safe), 0.0).reshape(-1, 1)


def _tc_head(h, W, b, degp):
    fi, fo = W.shape

    def body(h_ref, w_ref, b_ref, degp_ref, out_ref, u_ref):
        dinv = _dinv_block(degp_ref[...])
        hv = h_ref[...]
        out_ref[...] = (
            jnp.dot(hv, w_ref[...], preferred_element_type=jnp.float32,
                    precision=_PREC) + b_ref[...])
        u_ref[...] = hv * dinv

    return pl.pallas_call(
        body,
        grid=(GRID,),
        in_specs=[
            pl.BlockSpec((RB, fi), lambda i: (i, 0)),
            pl.BlockSpec((fi, fo), lambda i: (0, 0)),
            pl.BlockSpec((1, fo), lambda i: (0, 0)),
            pl.BlockSpec((2, RB), lambda i: (0, i)),
        ],
        out_specs=[
            pl.BlockSpec((RB, fo), lambda i: (i, 0)),
            pl.BlockSpec((RB, fi), lambda i: (i, 0)),
        ],
        out_shape=[
            jax.ShapeDtypeStruct((N_PAD, fo), jnp.float32),
            jax.ShapeDtypeStruct((N_PAD, fi), jnp.float32),
        ],
    )(h, W, b.reshape(1, -1), degp)


def _tc_mid(P, W, b, degp, out_in):
    fi, fo = W.shape

    def body(p_ref, w_ref, b_ref, degp_ref, oin_ref, out_ref, u_ref):
        dinv = _dinv_block(degp_ref[...])
        hk = (p_ref[0] + p_ref[1]) * dinv
        out_ref[...] = oin_ref[...] + (
            jnp.dot(hk, w_ref[...], preferred_element_type=jnp.float32,
                    precision=_PREC) + b_ref[...])
        u_ref[...] = hk * dinv

    return pl.pallas_call(
        body,
        grid=(GRID,),
        in_specs=[
            pl.BlockSpec((2, RB, fi), lambda i: (0, i, 0)),
            pl.BlockSpec((fi, fo), lambda i: (0, 0)),
            pl.BlockSpec((1, fo), lambda i: (0, 0)),
            pl.BlockSpec((2, RB), lambda i: (0, i)),
            pl.BlockSpec((RB, fo), lambda i: (i, 0)),
        ],
        out_specs=[
            pl.BlockSpec((RB, fo), lambda i: (i, 0)),
            pl.BlockSpec((RB, fi), lambda i: (i, 0)),
        ],
        out_shape=[
            jax.ShapeDtypeStruct((N_PAD, fo), jnp.float32),
            jax.ShapeDtypeStruct((N_PAD, fi), jnp.float32),
        ],
    )(P, W, b.reshape(1, -1), degp, out_in)


def _tc_bridge(P, W, b, degp, out_in, Wn, bn):
    fi, fo = W.shape
    fon = Wn.shape[1]

    def body(p_ref, w_ref, b_ref, degp_ref, oin_ref, wn_ref, bn_ref,
             out_ref, u_ref):
        dinv = _dinv_block(degp_ref[...])
        hk = (p_ref[0] + p_ref[1]) * dinv
        hp = jax.nn.relu(
            oin_ref[...] + jnp.dot(hk, w_ref[...],
                                   preferred_element_type=jnp.float32,
                                   precision=_PREC) + b_ref[...])
        out_ref[...] = (
            jnp.dot(hp, wn_ref[...], preferred_element_type=jnp.float32,
                    precision=_PREC) + bn_ref[...])
        u_ref[...] = hp * dinv

    return pl.pallas_call(
        body,
        grid=(GRID,),
        in_specs=[
            pl.BlockSpec((2, RB, fi), lambda i: (0, i, 0)),
            pl.BlockSpec((fi, fo), lambda i: (0, 0)),
            pl.BlockSpec((1, fo), lambda i: (0, 0)),
            pl.BlockSpec((2, RB), lambda i: (0, i)),
            pl.BlockSpec((RB, fo), lambda i: (i, 0)),
            pl.BlockSpec((fo, fon), lambda i: (0, 0)),
            pl.BlockSpec((1, fon), lambda i: (0, 0)),
        ],
        out_specs=[
            pl.BlockSpec((RB, fon), lambda i: (i, 0)),
            pl.BlockSpec((RB, fo), lambda i: (i, 0)),
        ],
        out_shape=[
            jax.ShapeDtypeStruct((N_PAD, fon), jnp.float32),
            jax.ShapeDtypeStruct((N_PAD, fo), jnp.float32),
        ],
    )(P, W, b.reshape(1, -1), degp, out_in, Wn, bn.reshape(1, -1))


def _tc_bridge_d(P, W, b, degp, out_in, Wn, bn):
    fi, fo = W.shape
    fon = Wn.shape[1]

    def body(p_ref, w_ref, b_ref, degp_ref, oin_ref, wn_ref, bn_ref,
             out_ref, u0_ref, u1_ref):
        dinv = _dinv_block(degp_ref[...])
        hk = (p_ref[0] + p_ref[1]) * dinv
        hp = jax.nn.relu(
            oin_ref[...] + jnp.dot(hk, w_ref[...],
                                   preferred_element_type=jnp.float32,
                                   precision=_PREC) + b_ref[...])
        out_ref[...] = (
            jnp.dot(hp, wn_ref[...], preferred_element_type=jnp.float32,
                    precision=_PREC) + bn_ref[...])
        us = hp * dinv
        u0_ref[...] = us[:, :fo // 2]
        u1_ref[...] = us[:, fo // 2:]

    return pl.pallas_call(
        body,
        grid=(GRID,),
        in_specs=[
            pl.BlockSpec((2, RB, fi), lambda i: (0, i, 0)),
            pl.BlockSpec((fi, fo), lambda i: (0, 0)),
            pl.BlockSpec((1, fo), lambda i: (0, 0)),
            pl.BlockSpec((2, RB), lambda i: (0, i)),
            pl.BlockSpec((RB, fo), lambda i: (i, 0)),
            pl.BlockSpec((fo, fon), lambda i: (0, 0)),
            pl.BlockSpec((1, fon), lambda i: (0, 0)),
        ],
        out_specs=[
            pl.BlockSpec((RB, fon), lambda i: (i, 0)),
            pl.BlockSpec((RB, fo // 2), lambda i: (i, 0)),
            pl.BlockSpec((RB, fo // 2), lambda i: (i, 0)),
        ],
        out_shape=[
            jax.ShapeDtypeStruct((N_PAD, fon), jnp.float32),
            jax.ShapeDtypeStruct((N_PAD, fo // 2), jnp.float32),
            jax.ShapeDtypeStruct((N_PAD, fo // 2), jnp.float32),
        ],
    )(P, W, b.reshape(1, -1), degp, out_in, Wn, bn.reshape(1, -1))


def _tc_mid_d(Pd, W, b, degp, out_in):
    fi, fo = W.shape
    ch = fi // 2

    def body(p_ref, w_ref, b_ref, degp_ref, oin_ref,
             out_ref, u0_ref, u1_ref):
        dinv = _dinv_block(degp_ref[...])
        s = p_ref[0] + p_ref[1]
        hk = jnp.concatenate([s[0], s[1]], axis=-1) * dinv
        out_ref[...] = oin_ref[...] + (
            jnp.dot(hk, w_ref[...], preferred_element_type=jnp.float32,
                    precision=_PREC) + b_ref[...])
        us = hk * dinv
        u0_ref[...] = us[:, :ch]
        u1_ref[...] = us[:, ch:]

    return pl.pallas_call(
        body,
        grid=(GRID,),
        in_specs=[
            pl.BlockSpec((2, 2, RB, ch), lambda i: (0, 0, i, 0)),
            pl.BlockSpec((fi, fo), lambda i: (0, 0)),
            pl.BlockSpec((1, fo), lambda i: (0, 0)),
            pl.BlockSpec((2, RB), lambda i: (0, i)),
            pl.BlockSpec((RB, fo), lambda i: (i, 0)),
        ],
        out_specs=[
            pl.BlockSpec((RB, fo), lambda i: (i, 0)),
            pl.BlockSpec((RB, ch), lambda i: (i, 0)),
            pl.BlockSpec((RB, ch), lambda i: (i, 0)),
        ],
        out_shape=[
            jax.ShapeDtypeStruct((N_PAD, fo), jnp.float32),
            jax.ShapeDtypeStruct((N_PAD, ch), jnp.float32),
            jax.ShapeDtypeStruct((N_PAD, ch), jnp.float32),
        ],
    )(Pd, W, b.reshape(1, -1), degp, out_in)


def _tc_tail_d(Pd, W, b, degp, out_in, fcW, fcb):
    fi, fo = W.shape
    ch = fi // 2
    fon = fcW.shape[1]

    def body(p_ref, w_ref, b_ref, degp_ref, oin_ref, wn_ref, bn_ref,
             out_ref):
        dinv = _dinv_block(degp_ref[...])
        s = p_ref[0] + p_ref[1]
        hk = jnp.concatenate([s[0], s[1]], axis=-1) * dinv
        hp = jax.nn.relu(
            oin_ref[...] + jnp.dot(hk, w_ref[...],
                                   preferred_element_type=jnp.float32,
                                   precision=_PREC) + b_ref[...])
        z = (jnp.dot(hp, wn_ref[...], preferred_element_type=jnp.float32,
                     precision=_PREC) + bn_ref[...])
        nrm = jnp.sqrt(jnp.sum(z * z, axis=-1, keepdims=True))
        out_ref[...] = z / jnp.maximum(nrm, 1e-12)

    return pl.pallas_call(
        body,
        grid=(GRID,),
        in_specs=[
            pl.BlockSpec((2, 2, RB, ch), lambda i: (0, 0, i, 0)),
            pl.BlockSpec((fi, fo), lambda i: (0, 0)),
            pl.BlockSpec((1, fo), lambda i: (0, 0)),
            pl.BlockSpec((2, RB), lambda i: (0, i)),
            pl.BlockSpec((RB, fo), lambda i: (i, 0)),
            pl.BlockSpec((fo, fon), lambda i: (0, 0)),
            pl.BlockSpec((1, fon), lambda i: (0, 0)),
        ],
        out_specs=pl.BlockSpec((RB, fon), lambda i: (i, 0)),
        out_shape=jax.ShapeDtypeStruct((N_PAD, fon), jnp.float32),
    )(Pd, W, b.reshape(1, -1), degp, out_in, fcW, fcb.reshape(1, -1))


def _tc_tail(P, W, b, degp, out_in, fcW, fcb):
    fi, fo = W.shape
    fon = fcW.shape[1]

    def body(p_ref, w_ref, b_ref, degp_ref, oin_ref, wn_ref, bn_ref, out_ref):
        dinv = _dinv_block(degp_ref[...])
        hk = (p_ref[0] + p_ref[1]) * dinv
        hp = jax.nn.relu(
            oin_ref[...] + jnp.dot(hk, w_ref[...],
                                   preferred_element_type=jnp.float32,
                                   precision=_PREC) + b_ref[...])
        z = (jnp.dot(hp, wn_ref[...], preferred_element_type=jnp.float32,
                     precision=_PREC) + bn_ref[...])
        nrm = jnp.sqrt(jnp.sum(z * z, axis=-1, keepdims=True))
        out_ref[...] = z / jnp.maximum(nrm, 1e-12)

    return pl.pallas_call(
        body,
        grid=(GRID,),
        in_specs=[
            pl.BlockSpec((2, RB, fi), lambda i: (0, i, 0)),
            pl.BlockSpec((fi, fo), lambda i: (0, 0)),
            pl.BlockSpec((1, fo), lambda i: (0, 0)),
            pl.BlockSpec((2, RB), lambda i: (0, i)),
            pl.BlockSpec((RB, fo), lambda i: (i, 0)),
            pl.BlockSpec((fo, fon), lambda i: (0, 0)),
            pl.BlockSpec((1, fon), lambda i: (0, 0)),
        ],
        out_specs=pl.BlockSpec((RB, fon), lambda i: (i, 0)),
        out_shape=jax.ShapeDtypeStruct((N_PAD, fon), jnp.float32),
    )(P, W, b.reshape(1, -1), degp, out_in, fcW, fcb.reshape(1, -1))


def kernel(x, edge_index, edge_attr,
           W1_0, b1_0, W1_1, b1_1,
           W2_0, b2_0, W2_1, b2_1, W2_2, b2_2,
           W3_0, b3_0, W3_1, b3_1, W3_2, b3_2, W3_3, b3_3,
           fc_W, fc_b):
    npad = E_PAD - E
    pad_nodes = (jnp.arange(npad, dtype=jnp.int32) % (N_PAD - N)) + N
    row3 = jnp.concatenate([edge_index[0], pad_nodes]).reshape(
        NW, NCHUNK, CHUNK)
    col3 = jnp.concatenate([edge_index[1], pad_nodes]).reshape(
        NW, NCHUNK, CHUNK)
    w3 = jnp.concatenate(
        [edge_attr, jnp.zeros((npad,), jnp.float32)]).reshape(
        NW, NCHUNK, CHUNK)
    x_p = jnp.pad(x, ((0, N_PAD - N), (0, 0)))

    row2 = row3.reshape(-1, CHUNK)
    col2 = col3.reshape(-1, CHUNK)
    w2 = w3.reshape(-1, CHUNK)

    degp = _deg_kernel(col3, w3)

    out0, u = _tc_head(x_p, W1_0, b1_0, degp)
    P = _prop_ring(u, row2, col2, w2, 128, 2)
    out0, u = _tc_bridge(P, W1_1, b1_1, degp, out0, W2_0, b2_0)
    P = _prop(u, row3, col3, w3, 32)
    out0, u = _tc_mid(P, W2_1, b2_1, degp, out0)
    P = _prop(u, row3, col3, w3, 32)
    out0, u0, u1 = _tc_bridge_d(P, W2_2, b2_2, degp, out0, W3_0, b3_0)
    Pd = _prop_dual(u0, u1, row2, col2, w2)
    out0, u0, u1 = _tc_mid_d(Pd, W3_1, b3_1, degp, out0)
    Pd = _prop_dual(u0, u1, row2, col2, w2)
    out0, u0, u1 = _tc_mid_d(Pd, W3_2, b3_2, degp, out0)
    Pd = _prop_dual(u0, u1, row2, col2, w2)
    final = _tc_tail_d(Pd, W3_3, b3_3, degp, out0, fc_W, fc_b)
    return final[:N]
